# Initial kernel scaffold; baseline (speedup 1.0000x reference)
import functools

import jax
import jax.numpy as jnp
from jax import lax
from jax.experimental import pallas as pl
from jax.experimental.pallas import tpu as pltpu
from jax.experimental.pallas import tpu_sc as plsc

N = 10000
E = 160000
T = 12
C = 128

NC, NS, L = 2, 16, 16
NW = NC * NS
N_PAD = 10240
STRIPE = N_PAD // NS
EK = 128
E_ALL = E + N
E_PAD = 172032
CH_TOTAL = E_PAD // EK
CH_PER_W = CH_TOTAL // NW
CH_PER_S = CH_TOTAL // NS
T_PER_CORE = T // NC
R_OUT = N_PAD * T

_MESH = plsc.VectorSubcoreMesh(core_axis_name="c", subcore_axis_name="s")


def _mm(x, w):
    R = x.shape[0]
    BM = 960
    assert R % BM == 0

    def body(x_ref, w_ref, o_ref):
        o_ref[...] = lax.dot_general(
            x_ref[...], w_ref[...], (((1,), (1,)), ((), ())),
            preferred_element_type=jnp.float32)

    return pl.pallas_call(
        body,
        grid=(R // BM,),
        in_specs=[
            pl.BlockSpec((BM, C), lambda i: (i, 0)),
            pl.BlockSpec((C, C), lambda i: (0, 0)),
        ],
        out_specs=pl.BlockSpec((BM, C), lambda i: (i, 0)),
        out_shape=jax.ShapeDtypeStruct((R, C), jnp.float32),
    )(x, w)


@functools.partial(
    pl.kernel,
    out_type=jax.ShapeDtypeStruct((E_PAD,), jnp.float32),
    mesh=_MESH,
    scratch_types=[
        pltpu.VMEM_SHARED((N_PAD,), jnp.float32),
        pltpu.VMEM_SHARED((N_PAD,), jnp.float32),
        pltpu.VMEM((STRIPE,), jnp.float32),
        pltpu.VMEM((N_PAD,), jnp.float32),
        pltpu.VMEM((EK,), jnp.int32),
        pltpu.VMEM((EK,), jnp.int32),
        pltpu.VMEM((EK,), jnp.float32),
        pltpu.VMEM((EK,), jnp.float32),
    ],
)
def _norm_kernel(row_hbm, col_hbm, ew_hbm, norm_hbm,
                 deg_sh, dis_sh, degb, disfull, rowb, colb, ewb, normb):
    c = lax.axis_index("c")
    s = lax.axis_index("s")
    wid = s * NC + c

    def zero_body(i, _):
        degb[pl.ds(i * L, L)] = jnp.zeros((L,), jnp.float32)
        return 0
    lax.fori_loop(0, STRIPE // L, zero_body, 0)
    pltpu.sync_copy(degb, deg_sh.at[pl.ds(s * STRIPE, STRIPE)])
    plsc.subcore_barrier()

    def deg_chunk(ci, _):
        base = ci * EK
        pltpu.sync_copy(col_hbm.at[pl.ds(base, EK)], colb)
        pltpu.sync_copy(ew_hbm.at[pl.ds(base, EK)], ewb)
        pltpu.sync_copy(ewb, deg_sh.at[colb], add=True)
        return 0
    lax.fori_loop(s * CH_PER_W, (s + 1) * CH_PER_W, deg_chunk, 0)
    lax.fori_loop((s + NS) * CH_PER_W, (s + NS + 1) * CH_PER_W, deg_chunk, 0)
    plsc.subcore_barrier()

    pltpu.sync_copy(deg_sh.at[pl.ds(s * STRIPE, STRIPE)], degb)

    def rsqrt_body(i, _):
        sl = pl.ds(i * L, L)
        x = degb[sl]
        xi = plsc.bitcast(x, jnp.int32)
        yi = jnp.int32(0x5F3759DF) - (xi >> 1)
        y = plsc.bitcast(yi, jnp.float32)
        hx = x * 0.5
        for _ in range(3):
            y = y * (1.5 - hx * y * y)
        degb[sl] = y
        return 0
    lax.fori_loop(0, STRIPE // L, rsqrt_body, 0)
    pltpu.sync_copy(degb, dis_sh.at[pl.ds(s * STRIPE, STRIPE)])
    plsc.subcore_barrier()

    pltpu.sync_copy(dis_sh, disfull)

    def norm_chunk(ci, _):
        base = ci * EK
        pltpu.sync_copy(row_hbm.at[pl.ds(base, EK)], rowb)
        pltpu.sync_copy(col_hbm.at[pl.ds(base, EK)], colb)
        pltpu.sync_copy(ew_hbm.at[pl.ds(base, EK)], ewb)
        for j in range(EK // L):
            sl = pl.ds(j * L, L)
            dr = plsc.load_gather(disfull, [rowb[sl]])
            dc = plsc.load_gather(disfull, [colb[sl]])
            normb[sl] = dr * ewb[sl] * dc
        pltpu.sync_copy(normb, norm_hbm.at[pl.ds(base, EK)])
        return 0
    lax.fori_loop(wid * CH_PER_W, (wid + 1) * CH_PER_W, norm_chunk, 0)


def _make_prop(act_kind):
    def body(xs_hbm, row_hbm, col_hbm, norm_hbm, b_hbm, out_hbm,
             acc, gbuf, zbuf, rowb, colb, gidxb, normb, oidxb, biasb, sem):
        c = lax.axis_index("c")
        s = lax.axis_index("s")
        pltpu.sync_copy(b_hbm, biasb)

        def zb_body(i, _):
            for j in range(C // L):
                zbuf[i, pl.ds(j * L, L)] = jnp.zeros((L,), jnp.float32)
            return 0
        lax.fori_loop(0, EK, zb_body, 0)

        iot = lax.iota(jnp.int32, L)

        def slice_body(ts, _):
            t = c * T_PER_CORE + ts
            for k in range(STRIPE // EK):
                pltpu.sync_copy(zbuf, acc.at[pl.ds(s * STRIPE + k * EK, EK)])
            plsc.subcore_barrier()

            def chunk(ci, _):
                base = (s * CH_PER_S + ci) * EK
                pltpu.sync_copy(row_hbm.at[pl.ds(base, EK)], rowb)
                pltpu.sync_copy(col_hbm.at[pl.ds(base, EK)], colb)
                pltpu.sync_copy(norm_hbm.at[pl.ds(base, EK)], normb)
                for j in range(EK // L):
                    sl = pl.ds(j * L, L)
                    gidxb[sl] = rowb[sl] * T + t
                pltpu.async_copy(xs_hbm.at[gidxb], gbuf, sem).wait()

                def scale(k, _):
                    sv = plsc.load_gather(normb, [jnp.zeros((L,), jnp.int32) + k])
                    for j in range(C // L):
                        sl = pl.ds(j * L, L)
                        gbuf[k, sl] = gbuf[k, sl] * sv
                    return 0
                lax.fori_loop(0, EK, scale, 0)
                pltpu.sync_copy(gbuf, acc.at[colb], add=True)
                return 0
            lax.fori_loop(0, CH_PER_S, chunk, 0)
            plsc.subcore_barrier()

            for k in range(STRIPE // EK):
                nbase = s * STRIPE + k * EK
                pltpu.sync_copy(acc.at[pl.ds(nbase, EK)], gbuf)
                for j in range(EK // L):
                    sl = pl.ds(j * L, L)
                    oidxb[sl] = (iot + (nbase + j * L)) * T + t

                def fin(r, _):
                    for j in range(C // L):
                        sl = pl.ds(j * L, L)
                        v = gbuf[r, sl] + biasb[sl]
                        if act_kind == "relu":
                            v = jnp.maximum(v, 0.0)
                        else:
                            v = 1.0 / (1.0 + jnp.exp(-v))
                        gbuf[r, sl] = v
                    return 0
                lax.fori_loop(0, EK, fin, 0)
                pltpu.sync_copy(gbuf, out_hbm.at[oidxb])
            return 0
        lax.fori_loop(0, T_PER_CORE, slice_body, 0)

    return pl.kernel(
        body,
        out_type=jax.ShapeDtypeStruct((R_OUT, C), jnp.float32),
        mesh=_MESH,
        scratch_types=[
            pltpu.VMEM_SHARED((N_PAD, C), jnp.float32),
            pltpu.VMEM((EK, C), jnp.float32),
            pltpu.VMEM((EK, C), jnp.float32),
            pltpu.VMEM((EK,), jnp.int32),
            pltpu.VMEM((EK,), jnp.int32),
            pltpu.VMEM((EK,), jnp.int32),
            pltpu.VMEM((EK,), jnp.float32),
            pltpu.VMEM((EK,), jnp.int32),
            pltpu.VMEM((C,), jnp.float32),
            pltpu.SemaphoreType.DMA,
        ],
    )


_prop_relu = _make_prop("relu")
_prop_sigmoid = _make_prop("sigmoid")


def kernel(X, edge_index, edge_weight, W1, b1, W2, b2):
    row = edge_index[0].astype(jnp.int32)
    col = edge_index[1].astype(jnp.int32)
    loop = jnp.arange(N, dtype=jnp.int32)
    npad = E_PAD - E_ALL
    padi = jnp.arange(npad, dtype=jnp.int32) % N
    row_all = jnp.concatenate([row, loop, padi])
    col_all = jnp.concatenate([col, loop, padi])
    ew_all = jnp.concatenate([
        edge_weight,
        jnp.ones((N,), jnp.float32),
        jnp.zeros((npad,), jnp.float32),
    ])

    norm = _norm_kernel(row_all, col_all, ew_all)

    x2d = X.reshape(N * T, C)
    xw1 = _mm(x2d, W1)
    a1 = _prop_relu(xw1, row_all, col_all, norm, b1)
    xw2 = _mm(a1, W2)
    a2 = _prop_sigmoid(xw2, row_all, col_all, norm, b2)

    out = a2.reshape(N_PAD, T, C)[:N]
    return out[None]


# speedup vs baseline: 5.8569x; 5.8569x over previous
import functools

import jax
import jax.numpy as jnp
from jax import lax
from jax.experimental import pallas as pl
from jax.experimental.pallas import tpu as pltpu
from jax.experimental.pallas import tpu_sc as plsc

N = 10000
E = 160000
T = 12
C = 128

NC, NS, L = 2, 16, 16
NW = NC * NS
N_PAD = 10240
STRIPE = N_PAD // NS
EK = 128
E_ALL = E + N
E_PAD = 172032
CH_TOTAL = E_PAD // EK
CH_PER_W = CH_TOTAL // NW
CH_PER_S = CH_TOTAL // NS
T_PER_CORE = T // NC
R_OUT = N_PAD * T

_MESH = plsc.VectorSubcoreMesh(core_axis_name="c", subcore_axis_name="s")
_SC_PARAMS = pltpu.CompilerParams(needs_layout_passes=False)


def _mm(x, w):
    R = x.shape[0]
    BM = 960
    assert R % BM == 0

    def body(x_ref, w_ref, o_ref):
        o_ref[...] = lax.dot_general(
            x_ref[...], w_ref[...], (((1,), (1,)), ((), ())),
            preferred_element_type=jnp.float32)

    return pl.pallas_call(
        body,
        grid=(R // BM,),
        in_specs=[
            pl.BlockSpec((BM, C), lambda i: (i, 0)),
            pl.BlockSpec((C, C), lambda i: (0, 0)),
        ],
        out_specs=pl.BlockSpec((BM, C), lambda i: (i, 0)),
        out_shape=jax.ShapeDtypeStruct((R, C), jnp.float32),
    )(x, w)


@functools.partial(
    pl.kernel,
    out_type=jax.ShapeDtypeStruct((E_PAD,), jnp.float32),
    mesh=_MESH,
    compiler_params=_SC_PARAMS,
    scratch_types=[
        pltpu.VMEM_SHARED((N_PAD,), jnp.float32),
        pltpu.VMEM_SHARED((N_PAD,), jnp.float32),
        pltpu.VMEM((STRIPE,), jnp.float32),
        pltpu.VMEM((N_PAD,), jnp.float32),
        pltpu.VMEM((EK,), jnp.int32),
        pltpu.VMEM((EK,), jnp.int32),
        pltpu.VMEM((EK,), jnp.float32),
        pltpu.VMEM((EK,), jnp.float32),
    ],
)
def _norm_kernel(row_hbm, col_hbm, ew_hbm, norm_hbm,
                 deg_sh, dis_sh, degb, disfull, rowb, colb, ewb, normb):
    c = lax.axis_index("c")
    s = lax.axis_index("s")
    wid = s * NC + c

    def zero_body(i, _):
        degb[pl.ds(i * L, L)] = jnp.zeros((L,), jnp.float32)
        return 0
    lax.fori_loop(0, STRIPE // L, zero_body, 0)
    pltpu.sync_copy(degb, deg_sh.at[pl.ds(s * STRIPE, STRIPE)])
    plsc.subcore_barrier()

    def deg_chunk(ci, _):
        base = ci * EK
        pltpu.sync_copy(col_hbm.at[pl.ds(base, EK)], colb)
        pltpu.sync_copy(ew_hbm.at[pl.ds(base, EK)], ewb)
        pltpu.sync_copy(ewb, deg_sh.at[colb], add=True)
        return 0
    lax.fori_loop(s * CH_PER_W, (s + 1) * CH_PER_W, deg_chunk, 0)
    lax.fori_loop((s + NS) * CH_PER_W, (s + NS + 1) * CH_PER_W, deg_chunk, 0)
    plsc.subcore_barrier()

    pltpu.sync_copy(deg_sh.at[pl.ds(s * STRIPE, STRIPE)], degb)

    def rsqrt_body(i, _):
        sl = pl.ds(i * L, L)
        x = degb[sl]
        xi = lax.bitcast_convert_type(x, jnp.int32)
        yi = jnp.int32(0x5F3759DF) - (xi >> 1)
        y = lax.bitcast_convert_type(yi, jnp.float32)
        hx = x * 0.5
        for _ in range(3):
            y = y * (1.5 - hx * y * y)
        degb[sl] = y
        return 0
    lax.fori_loop(0, STRIPE // L, rsqrt_body, 0)
    pltpu.sync_copy(degb, dis_sh.at[pl.ds(s * STRIPE, STRIPE)])
    plsc.subcore_barrier()

    pltpu.sync_copy(dis_sh, disfull)

    def norm_chunk(ci, _):
        base = ci * EK
        pltpu.sync_copy(row_hbm.at[pl.ds(base, EK)], rowb)
        pltpu.sync_copy(col_hbm.at[pl.ds(base, EK)], colb)
        pltpu.sync_copy(ew_hbm.at[pl.ds(base, EK)], ewb)
        for j in range(EK // L):
            sl = pl.ds(j * L, L)
            dr = plsc.load_gather(disfull, [rowb[sl]])
            dc = plsc.load_gather(disfull, [colb[sl]])
            normb[sl] = dr * ewb[sl] * dc
        pltpu.sync_copy(normb, norm_hbm.at[pl.ds(base, EK)])
        return 0
    lax.fori_loop(wid * CH_PER_W, (wid + 1) * CH_PER_W, norm_chunk, 0)


def _make_prop(act_kind):
    def body(xs_hbm, row_hbm, col_hbm, norm_hbm, b_hbm, out_hbm,
             acc, gbuf, zbuf, rowb, colb, gidxb, normb, oidxb, biasb, sem):
        c = lax.axis_index("c")
        s = lax.axis_index("s")
        pltpu.sync_copy(b_hbm, biasb)

        def zb_body(i, _):
            for j in range(C // L):
                zbuf[i, pl.ds(j * L, L)] = jnp.zeros((L,), jnp.float32)
            return 0
        lax.fori_loop(0, EK, zb_body, 0)

        iot = lax.iota(jnp.int32, L)

        def slice_body(ts, _):
            t = c * T_PER_CORE + ts
            for k in range(STRIPE // EK):
                pltpu.sync_copy(zbuf, acc.at[pl.ds(s * STRIPE + k * EK, EK)])
            plsc.subcore_barrier()

            def chunk(ci, _):
                base = (s * CH_PER_S + ci) * EK
                pltpu.sync_copy(row_hbm.at[pl.ds(base, EK)], rowb)
                pltpu.sync_copy(col_hbm.at[pl.ds(base, EK)], colb)
                pltpu.sync_copy(norm_hbm.at[pl.ds(base, EK)], normb)
                for j in range(EK // L):
                    sl = pl.ds(j * L, L)
                    gidxb[sl] = rowb[sl] * T + t
                pltpu.async_copy(xs_hbm.at[gidxb], gbuf, sem).wait()

                def scale(k, _):
                    sv = plsc.load_gather(normb, [jnp.zeros((L,), jnp.int32) + k])
                    for j in range(C // L):
                        sl = pl.ds(j * L, L)
                        gbuf[k, sl] = gbuf[k, sl] * sv
                    return 0
                lax.fori_loop(0, EK, scale, 0)
                pltpu.sync_copy(gbuf, acc.at[colb], add=True)
                return 0
            lax.fori_loop(0, CH_PER_S, chunk, 0)
            plsc.subcore_barrier()

            for k in range(STRIPE // EK):
                nbase = s * STRIPE + k * EK
                pltpu.sync_copy(acc.at[pl.ds(nbase, EK)], gbuf)
                for j in range(EK // L):
                    sl = pl.ds(j * L, L)
                    oidxb[sl] = (iot + (nbase + j * L)) * T + t

                def fin(r, _):
                    for j in range(C // L):
                        sl = pl.ds(j * L, L)
                        v = gbuf[r, sl] + biasb[sl]
                        if act_kind == "relu":
                            v = jnp.maximum(v, 0.0)
                        else:
                            v = 1.0 / (1.0 + jnp.exp(-v))
                        gbuf[r, sl] = v
                    return 0
                lax.fori_loop(0, EK, fin, 0)
                pltpu.sync_copy(gbuf, out_hbm.at[oidxb])
            return 0
        lax.fori_loop(0, T_PER_CORE, slice_body, 0)

    return pl.kernel(
        body,
        out_type=jax.ShapeDtypeStruct((R_OUT, C), jnp.float32),
        mesh=_MESH,
        compiler_params=_SC_PARAMS,
        scratch_types=[
            pltpu.VMEM_SHARED((N_PAD, C), jnp.float32),
            pltpu.VMEM((EK, C), jnp.float32),
            pltpu.VMEM((EK, C), jnp.float32),
            pltpu.VMEM((EK,), jnp.int32),
            pltpu.VMEM((EK,), jnp.int32),
            pltpu.VMEM((EK,), jnp.int32),
            pltpu.VMEM((EK,), jnp.float32),
            pltpu.VMEM((EK,), jnp.int32),
            pltpu.VMEM((C,), jnp.float32),
            pltpu.SemaphoreType.DMA,
        ],
    )


_prop_relu = _make_prop("relu")
_prop_sigmoid = _make_prop("sigmoid")


def kernel(X, edge_index, edge_weight, W1, b1, W2, b2):
    row = edge_index[0].astype(jnp.int32)
    col = edge_index[1].astype(jnp.int32)
    loop = jnp.arange(N, dtype=jnp.int32)
    npad = E_PAD - E_ALL
    padi = jnp.arange(npad, dtype=jnp.int32) % N
    row_all = jnp.concatenate([row, loop, padi])
    col_all = jnp.concatenate([col, loop, padi])
    ew_all = jnp.concatenate([
        edge_weight,
        jnp.ones((N,), jnp.float32),
        jnp.zeros((npad,), jnp.float32),
    ])

    norm = _norm_kernel(row_all, col_all, ew_all)

    x2d = X.reshape(N * T, C)
    xw1 = _mm(x2d, W1)
    a1 = _prop_relu(xw1, row_all, col_all, norm, b1)
    xw2 = _mm(a1, W2)
    a2 = _prop_sigmoid(xw2, row_all, col_all, norm, b2)

    out = a2.reshape(N_PAD, T, C)[:N]
    return out[None]


# speedup vs baseline: 8.9495x; 1.5280x over previous
import functools

import jax
import jax.numpy as jnp
from jax import lax
from jax.experimental import pallas as pl
from jax.experimental.pallas import tpu as pltpu
from jax.experimental.pallas import tpu_sc as plsc

N = 10000
E = 160000
T = 12
C = 128

NC, NS, L = 2, 16, 16
NW = NC * NS
N_PAD = 10240
STRIPE = N_PAD // NS
EK = 128
E_ALL = E + N
E_PAD = 172032
CH_TOTAL = E_PAD // EK
CH_PER_W = CH_TOTAL // NW
CH_PER_S = CH_TOTAL // NS
T_PER_CORE = T // NC
R_OUT = N_PAD * T

_MESH = plsc.VectorSubcoreMesh(core_axis_name="c", subcore_axis_name="s")
_SC_PARAMS = pltpu.CompilerParams(needs_layout_passes=False)


def _mm(x, w):
    R = x.shape[0]
    BM = 960
    assert R % BM == 0

    def body(x_ref, w_ref, o_ref):
        o_ref[...] = lax.dot_general(
            x_ref[...], w_ref[...], (((1,), (1,)), ((), ())),
            preferred_element_type=jnp.float32)

    return pl.pallas_call(
        body,
        grid=(R // BM,),
        in_specs=[
            pl.BlockSpec((BM, C), lambda i: (i, 0)),
            pl.BlockSpec((C, C), lambda i: (0, 0)),
        ],
        out_specs=pl.BlockSpec((BM, C), lambda i: (i, 0)),
        out_shape=jax.ShapeDtypeStruct((R, C), jnp.float32),
    )(x, w)


@functools.partial(
    pl.kernel,
    out_type=jax.ShapeDtypeStruct((E_PAD,), jnp.float32),
    mesh=_MESH,
    compiler_params=_SC_PARAMS,
    scratch_types=[
        pltpu.VMEM_SHARED((N_PAD,), jnp.float32),
        pltpu.VMEM_SHARED((N_PAD,), jnp.float32),
        pltpu.VMEM((STRIPE,), jnp.float32),
        pltpu.VMEM((N_PAD,), jnp.float32),
        pltpu.VMEM((EK,), jnp.int32),
        pltpu.VMEM((EK,), jnp.int32),
        pltpu.VMEM((EK,), jnp.float32),
        pltpu.VMEM((EK,), jnp.float32),
    ],
)
def _norm_kernel(row_hbm, col_hbm, ew_hbm, norm_hbm,
                 deg_sh, dis_sh, degb, disfull, rowb, colb, ewb, normb):
    c = lax.axis_index("c")
    s = lax.axis_index("s")
    wid = s * NC + c

    def zero_body(i, _):
        degb[pl.ds(i * L, L)] = jnp.zeros((L,), jnp.float32)
        return 0
    lax.fori_loop(0, STRIPE // L, zero_body, 0)
    pltpu.sync_copy(degb, deg_sh.at[pl.ds(s * STRIPE, STRIPE)])
    plsc.subcore_barrier()

    def deg_chunk(ci, _):
        base = ci * EK
        pltpu.sync_copy(col_hbm.at[pl.ds(base, EK)], colb)
        pltpu.sync_copy(ew_hbm.at[pl.ds(base, EK)], ewb)
        pltpu.sync_copy(ewb, deg_sh.at[colb], add=True)
        return 0
    lax.fori_loop(s * CH_PER_W, (s + 1) * CH_PER_W, deg_chunk, 0)
    lax.fori_loop((s + NS) * CH_PER_W, (s + NS + 1) * CH_PER_W, deg_chunk, 0)
    plsc.subcore_barrier()

    pltpu.sync_copy(deg_sh.at[pl.ds(s * STRIPE, STRIPE)], degb)

    def rsqrt_body(i, _):
        sl = pl.ds(i * L, L)
        x = degb[sl]
        xi = lax.bitcast_convert_type(x, jnp.int32)
        yi = jnp.int32(0x5F3759DF) - (xi >> 1)
        y = lax.bitcast_convert_type(yi, jnp.float32)
        hx = x * 0.5
        for _ in range(3):
            y = y * (1.5 - hx * y * y)
        degb[sl] = y
        return 0
    lax.fori_loop(0, STRIPE // L, rsqrt_body, 0)
    pltpu.sync_copy(degb, dis_sh.at[pl.ds(s * STRIPE, STRIPE)])
    plsc.subcore_barrier()

    pltpu.sync_copy(dis_sh, disfull)

    def norm_chunk(ci, _):
        base = ci * EK
        pltpu.sync_copy(row_hbm.at[pl.ds(base, EK)], rowb)
        pltpu.sync_copy(col_hbm.at[pl.ds(base, EK)], colb)
        pltpu.sync_copy(ew_hbm.at[pl.ds(base, EK)], ewb)
        for j in range(EK // L):
            sl = pl.ds(j * L, L)
            dr = plsc.load_gather(disfull, [rowb[sl]])
            dc = plsc.load_gather(disfull, [colb[sl]])
            normb[sl] = dr * ewb[sl] * dc
        pltpu.sync_copy(normb, norm_hbm.at[pl.ds(base, EK)])
        return 0
    lax.fori_loop(wid * CH_PER_W, (wid + 1) * CH_PER_W, norm_chunk, 0)


def _make_prop(act_kind):
    def body(xs_hbm, row_hbm, col_hbm, norm_hbm, b_hbm, out_hbm,
             acc, gbufA, gbufB, rowA, rowB, colA, colB, normA, normB,
             oidxb, biasb, isemA, isemB, gsemA, gsemB):
        c = lax.axis_index("c")
        s = lax.axis_index("s")
        pltpu.sync_copy(b_hbm, biasb)
        iot = lax.iota(jnp.int32, L)
        ebase = s * (CH_PER_S * EK)

        def istart(ci, rowb, colb, normb, isem):
            base = ebase + ci * EK
            pltpu.async_copy(row_hbm.at[pl.ds(base, EK)], rowb, isem)
            pltpu.async_copy(col_hbm.at[pl.ds(base, EK)], colb, isem)
            pltpu.async_copy(norm_hbm.at[pl.ds(base, EK)], normb, isem)

        def iwait(ci, rowb, colb, normb, isem):
            base = ebase + ci * EK
            pltpu.make_async_copy(row_hbm.at[pl.ds(base, EK)], rowb, isem).wait()
            pltpu.make_async_copy(col_hbm.at[pl.ds(base, EK)], colb, isem).wait()
            pltpu.make_async_copy(norm_hbm.at[pl.ds(base, EK)], normb, isem).wait()

        def scale(normb, gbuf):
            def sc4(q, _):
                for e in range(4):
                    k = q * 4 + e
                    sv = plsc.load_gather(normb, [jnp.zeros((L,), jnp.int32) + k])
                    for j in range(C // L):
                        sl = pl.ds(j * L, L)
                        gbuf[k, sl] = gbuf[k, sl] * sv
                return 0
            lax.fori_loop(0, EK // 4, sc4, 0)

        def slice_body(ts, _):
            t = c * T_PER_CORE + ts

            def gstart(t, rowb, gbuf, gsem):
                for j in range(EK // L):
                    sl = pl.ds(j * L, L)
                    rowb[sl] = rowb[sl] * T + t
                pltpu.async_copy(xs_hbm.at[rowb], gbuf, gsem)

            def gwait(rowb, gbuf, gsem):
                pltpu.make_async_copy(xs_hbm.at[rowb], gbuf, gsem).wait()

            def zb_body(i, _):
                for j in range(C // L):
                    gbufA[i, pl.ds(j * L, L)] = jnp.zeros((L,), jnp.float32)
                return 0
            lax.fori_loop(0, EK, zb_body, 0)
            for k in range(STRIPE // EK):
                pltpu.async_copy(
                    gbufA, acc.at[pl.ds(s * STRIPE + k * EK, EK)], gsemA)
            for k in range(STRIPE // EK):
                pltpu.make_async_copy(
                    gbufA, acc.at[pl.ds(s * STRIPE + k * EK, EK)], gsemA).wait()
            plsc.subcore_barrier()

            istart(0, rowA, colA, normA, isemA)
            istart(1, rowB, colB, normB, isemB)
            iwait(0, rowA, colA, normA, isemA)
            gstart(t, rowA, gbufA, gsemA)

            def pair(k, _):
                a = 2 * k
                b = 2 * k + 1
                last = k >= CH_PER_S // 2 - 1
                gwait(rowA, gbufA, gsemA)
                scale(normA, gbufA)
                iwait(b, rowB, colB, normB, isemB)
                gstart(t, rowB, gbufB, gsemB)
                pltpu.sync_copy(gbufA, acc.at[colA], add=True)

                @pl.when(jnp.logical_not(last))
                def _():
                    istart(a + 2, rowA, colA, normA, isemA)

                gwait(rowB, gbufB, gsemB)
                scale(normB, gbufB)

                @pl.when(jnp.logical_not(last))
                def _():
                    iwait(a + 2, rowA, colA, normA, isemA)
                    gstart(t, rowA, gbufA, gsemA)

                pltpu.sync_copy(gbufB, acc.at[colB], add=True)

                @pl.when(jnp.logical_not(last))
                def _():
                    istart(b + 2, rowB, colB, normB, isemB)
                return 0
            lax.fori_loop(0, CH_PER_S // 2, pair, 0)
            plsc.subcore_barrier()

            for k in range(STRIPE // EK):
                nbase = s * STRIPE + k * EK
                pltpu.sync_copy(acc.at[pl.ds(nbase, EK)], gbufA)
                for j in range(EK // L):
                    sl = pl.ds(j * L, L)
                    oidxb[sl] = (iot + (nbase + j * L)) * T + t

                def fin(r, _):
                    for j in range(C // L):
                        sl = pl.ds(j * L, L)
                        v = gbufA[r, sl] + biasb[sl]
                        if act_kind == "relu":
                            v = jnp.maximum(v, 0.0)
                        else:
                            v = 1.0 / (1.0 + jnp.exp(-v))
                        gbufA[r, sl] = v
                    return 0
                lax.fori_loop(0, EK, fin, 0)
                pltpu.sync_copy(gbufA, out_hbm.at[oidxb])
            return 0
        lax.fori_loop(0, T_PER_CORE, slice_body, 0)

    return pl.kernel(
        body,
        out_type=jax.ShapeDtypeStruct((R_OUT, C), jnp.float32),
        mesh=_MESH,
        compiler_params=_SC_PARAMS,
        scratch_types=[
            pltpu.VMEM_SHARED((N_PAD, C), jnp.float32),
            pltpu.VMEM((EK, C), jnp.float32),
            pltpu.VMEM((EK, C), jnp.float32),
            pltpu.VMEM((EK,), jnp.int32),
            pltpu.VMEM((EK,), jnp.int32),
            pltpu.VMEM((EK,), jnp.int32),
            pltpu.VMEM((EK,), jnp.int32),
            pltpu.VMEM((EK,), jnp.float32),
            pltpu.VMEM((EK,), jnp.float32),
            pltpu.VMEM((EK,), jnp.int32),
            pltpu.VMEM((C,), jnp.float32),
            pltpu.SemaphoreType.DMA,
            pltpu.SemaphoreType.DMA,
            pltpu.SemaphoreType.DMA,
            pltpu.SemaphoreType.DMA,
        ],
    )


_prop_relu = _make_prop("relu")
_prop_sigmoid = _make_prop("sigmoid")


def kernel(X, edge_index, edge_weight, W1, b1, W2, b2):
    row = edge_index[0].astype(jnp.int32)
    col = edge_index[1].astype(jnp.int32)
    loop = jnp.arange(N, dtype=jnp.int32)
    npad = E_PAD - E_ALL
    padi = jnp.arange(npad, dtype=jnp.int32) % N
    row_all = jnp.concatenate([row, loop, padi])
    col_all = jnp.concatenate([col, loop, padi])
    ew_all = jnp.concatenate([
        edge_weight,
        jnp.ones((N,), jnp.float32),
        jnp.zeros((npad,), jnp.float32),
    ])

    norm = _norm_kernel(row_all, col_all, ew_all)

    x2d = X.reshape(N * T, C)
    xw1 = _mm(x2d, W1)
    a1 = _prop_relu(xw1, row_all, col_all, norm, b1)
    xw2 = _mm(a1, W2)
    a2 = _prop_sigmoid(xw2, row_all, col_all, norm, b2)

    out = a2.reshape(N_PAD, T, C)[:N]
    return out[None]


# speedup vs baseline: 12.1024x; 1.3523x over previous
import functools

import jax
import jax.numpy as jnp
from jax import lax
from jax.experimental import pallas as pl
from jax.experimental.pallas import tpu as pltpu
from jax.experimental.pallas import tpu_sc as plsc

N = 10000
E = 160000
T = 12
C = 128

NC, NS, L = 2, 16, 16
NW = NC * NS
N_PAD = 10240
STRIPE = N_PAD // NS
E_ALL = E + N
E_PAD = 172032
EK = 128
CH_PER_W = E_PAD // (NW * EK)
EKP = 112
CHP = E_PAD // (NS * EKP)
FIN = 64
T_PER_CORE = T // NC
R_OUT = N_PAD * T

_MESH = plsc.VectorSubcoreMesh(core_axis_name="c", subcore_axis_name="s")
_SC_PARAMS = pltpu.CompilerParams(needs_layout_passes=False)


def _mm(x, w):
    R = x.shape[0]
    BM = 960
    assert R % BM == 0

    def body(x_ref, w_ref, o_ref):
        o_ref[...] = lax.dot_general(
            x_ref[...], w_ref[...], (((1,), (1,)), ((), ())),
            preferred_element_type=jnp.float32)

    return pl.pallas_call(
        body,
        grid=(R // BM,),
        in_specs=[
            pl.BlockSpec((BM, C), lambda i: (i, 0)),
            pl.BlockSpec((C, C), lambda i: (0, 0)),
        ],
        out_specs=pl.BlockSpec((BM, C), lambda i: (i, 0)),
        out_shape=jax.ShapeDtypeStruct((R, C), jnp.float32),
    )(x, w)


@functools.partial(
    pl.kernel,
    out_type=jax.ShapeDtypeStruct((E_PAD,), jnp.float32),
    mesh=_MESH,
    compiler_params=_SC_PARAMS,
    scratch_types=[
        pltpu.VMEM_SHARED((N_PAD,), jnp.float32),
        pltpu.VMEM_SHARED((N_PAD,), jnp.float32),
        pltpu.VMEM((STRIPE,), jnp.float32),
        pltpu.VMEM((N_PAD,), jnp.float32),
        pltpu.VMEM((EK,), jnp.int32),
        pltpu.VMEM((EK,), jnp.int32),
        pltpu.VMEM((EK,), jnp.float32),
        pltpu.VMEM((EK,), jnp.float32),
    ],
)
def _norm_kernel(row_hbm, col_hbm, ew_hbm, norm_hbm,
                 deg_sh, dis_sh, degb, disfull, rowb, colb, ewb, normb):
    c = lax.axis_index("c")
    s = lax.axis_index("s")
    wid = s * NC + c

    def zero_body(i, _):
        degb[pl.ds(i * L, L)] = jnp.zeros((L,), jnp.float32)
        return 0
    lax.fori_loop(0, STRIPE // L, zero_body, 0)
    pltpu.sync_copy(degb, deg_sh.at[pl.ds(s * STRIPE, STRIPE)])
    plsc.subcore_barrier()

    def deg_chunk(ci, _):
        base = ci * EK
        pltpu.sync_copy(col_hbm.at[pl.ds(base, EK)], colb)
        pltpu.sync_copy(ew_hbm.at[pl.ds(base, EK)], ewb)
        pltpu.sync_copy(ewb, deg_sh.at[colb], add=True)
        return 0
    lax.fori_loop(s * CH_PER_W, (s + 1) * CH_PER_W, deg_chunk, 0)
    lax.fori_loop((s + NS) * CH_PER_W, (s + NS + 1) * CH_PER_W, deg_chunk, 0)
    plsc.subcore_barrier()

    pltpu.sync_copy(deg_sh.at[pl.ds(s * STRIPE, STRIPE)], degb)

    def rsqrt_body(i, _):
        sl = pl.ds(i * L, L)
        x = degb[sl]
        xi = lax.bitcast_convert_type(x, jnp.int32)
        yi = jnp.int32(0x5F3759DF) - (xi >> 1)
        y = lax.bitcast_convert_type(yi, jnp.float32)
        hx = x * 0.5
        for _ in range(3):
            y = y * (1.5 - hx * y * y)
        degb[sl] = y
        return 0
    lax.fori_loop(0, STRIPE // L, rsqrt_body, 0)
    pltpu.sync_copy(degb, dis_sh.at[pl.ds(s * STRIPE, STRIPE)])
    plsc.subcore_barrier()

    pltpu.sync_copy(dis_sh, disfull)

    def norm_chunk(ci, _):
        base = ci * EK
        pltpu.sync_copy(row_hbm.at[pl.ds(base, EK)], rowb)
        pltpu.sync_copy(col_hbm.at[pl.ds(base, EK)], colb)
        pltpu.sync_copy(ew_hbm.at[pl.ds(base, EK)], ewb)
        for j in range(EK // L):
            sl = pl.ds(j * L, L)
            dr = plsc.load_gather(disfull, [rowb[sl]])
            dc = plsc.load_gather(disfull, [colb[sl]])
            normb[sl] = dr * ewb[sl] * dc
        pltpu.sync_copy(normb, norm_hbm.at[pl.ds(base, EK)])
        return 0
    lax.fori_loop(wid * CH_PER_W, (wid + 1) * CH_PER_W, norm_chunk, 0)


def _make_prop(act_kind):
    def body(xs_hbm, row_hbm, col_hbm, norm_hbm, b_hbm, out_hbm,
             acc, gbuf0, gbuf1, gbuf2,
             row0, row1, row2, row3, col0, col1, col2, col3,
             norm0, norm1, norm2, norm3,
             oidxb, biasb, isem0, isem1, isem2, isem3,
             gsem0, gsem1, gsem2, ssem0, ssem1, ssem2):
        c = lax.axis_index("c")
        s = lax.axis_index("s")
        pltpu.sync_copy(b_hbm, biasb)
        iot = lax.iota(jnp.int32, L)
        ebase = s * (CHP * EKP)
        P = [(row0, col0, norm0, isem0), (row1, col1, norm1, isem1),
             (row2, col2, norm2, isem2), (row3, col3, norm3, isem3)]
        G = [(gbuf0, gsem0, ssem0), (gbuf1, gsem1, ssem1),
             (gbuf2, gsem2, ssem2)]

        def istart(ci, p):
            rowb, colb, normb, isem = p
            base = ebase + ci * EKP
            pltpu.async_copy(row_hbm.at[pl.ds(base, EKP)], rowb, isem)
            pltpu.async_copy(col_hbm.at[pl.ds(base, EKP)], colb, isem)
            pltpu.async_copy(norm_hbm.at[pl.ds(base, EKP)], normb, isem)

        def iwait(ci, p):
            rowb, colb, normb, isem = p
            base = ebase + ci * EKP
            pltpu.make_async_copy(row_hbm.at[pl.ds(base, EKP)], rowb, isem).wait()
            pltpu.make_async_copy(col_hbm.at[pl.ds(base, EKP)], colb, isem).wait()
            pltpu.make_async_copy(norm_hbm.at[pl.ds(base, EKP)], normb, isem).wait()

        def scale(p, g):
            normb, gbuf = p[2], g[0]

            def sc4(q, _):
                for e in range(4):
                    k = q * 4 + e
                    sv = plsc.load_gather(normb, [jnp.zeros((L,), jnp.int32) + k])
                    for j in range(C // L):
                        sl = pl.ds(j * L, L)
                        gbuf[k, sl] = gbuf[k, sl] * sv
                return 0
            lax.fori_loop(0, EKP // 4, sc4, 0)

        def slice_body(ts, _):
            t = c * T_PER_CORE + ts

            def gstart(p, g):
                rowb = p[0]
                for j in range(EKP // L):
                    sl = pl.ds(j * L, L)
                    rowb[sl] = rowb[sl] * T + t
                pltpu.async_copy(xs_hbm.at[rowb], g[0], g[1])

            def gwait(p, g):
                pltpu.make_async_copy(xs_hbm.at[p[0]], g[0], g[1]).wait()

            def sstart(p, g):
                pltpu.async_copy(g[0], acc.at[p[1]], g[2], add=True)

            def swait(p, g):
                pltpu.make_async_copy(g[0], acc.at[p[1]], g[2]).wait()

            def zb_body(i, _):
                for j in range(C // L):
                    gbuf0[i, pl.ds(j * L, L)] = jnp.zeros((L,), jnp.float32)
                return 0
            lax.fori_loop(0, FIN, zb_body, 0)
            for k in range(STRIPE // FIN):
                pltpu.async_copy(
                    gbuf0.at[pl.ds(0, FIN)],
                    acc.at[pl.ds(s * STRIPE + k * FIN, FIN)], gsem0)
            for k in range(STRIPE // FIN):
                pltpu.make_async_copy(
                    gbuf0.at[pl.ds(0, FIN)],
                    acc.at[pl.ds(s * STRIPE + k * FIN, FIN)], gsem0).wait()
            plsc.subcore_barrier()

            istart(0, P[0])
            istart(1, P[1])
            istart(2, P[2])
            iwait(0, P[0])
            gstart(P[0], G[0])
            iwait(1, P[1])
            gstart(P[1], G[1])

            NU = 12
            NIT = CHP // NU

            def run(k, _):
                base = NU * k
                for q in range(NU):
                    i = base + q
                    p, g = P[q % 4], G[q % 3]
                    pm1, gm1 = P[(q - 1) % 4], G[(q - 1) % 3]
                    p2 = P[(q + 2) % 4]
                    gwait(p, g)
                    scale(p, g)
                    sstart(p, g)
                    if q == 0:
                        @pl.when(k > 0)
                        def _(pm1=pm1, gm1=gm1):
                            swait(pm1, gm1)
                        istart(i + 3, pm1)
                        iwait(i + 2, p2)
                        gstart(p2, gm1)
                    else:
                        swait(pm1, gm1)
                        if q <= 8:
                            istart(i + 3, pm1)
                        else:
                            @pl.when(k < NIT - 1)
                            def _(i=i, pm1=pm1):
                                istart(i + 3, pm1)
                        if q <= 9:
                            iwait(i + 2, p2)
                            gstart(p2, gm1)
                        else:
                            @pl.when(k < NIT - 1)
                            def _(i=i, p2=p2, gm1=gm1):
                                iwait(i + 2, p2)
                                gstart(p2, gm1)
                return 0
            lax.fori_loop(0, NIT, run, 0)
            swait(P[3], G[2])
            plsc.subcore_barrier()

            for k in range(STRIPE // FIN):
                nbase = s * STRIPE + k * FIN
                pltpu.sync_copy(acc.at[pl.ds(nbase, FIN)],
                                gbuf0.at[pl.ds(0, FIN)])
                for j in range(FIN // L):
                    sl = pl.ds(j * L, L)
                    oidxb[sl] = (iot + (nbase + j * L)) * T + t

                def fin(r, _):
                    for j in range(C // L):
                        sl = pl.ds(j * L, L)
                        v = gbuf0[r, sl] + biasb[sl]
                        if act_kind == "relu":
                            v = jnp.maximum(v, 0.0)
                        else:
                            v = 1.0 / (1.0 + jnp.exp(-v))
                        gbuf0[r, sl] = v
                    return 0
                lax.fori_loop(0, FIN, fin, 0)
                pltpu.sync_copy(gbuf0.at[pl.ds(0, FIN)], out_hbm.at[oidxb])
            return 0
        lax.fori_loop(0, T_PER_CORE, slice_body, 0)

    return pl.kernel(
        body,
        out_type=jax.ShapeDtypeStruct((R_OUT, C), jnp.float32),
        mesh=_MESH,
        compiler_params=_SC_PARAMS,
        scratch_types=[
            pltpu.VMEM_SHARED((N_PAD, C), jnp.float32),
            pltpu.VMEM((EKP, C), jnp.float32),
            pltpu.VMEM((EKP, C), jnp.float32),
            pltpu.VMEM((EKP, C), jnp.float32),
            pltpu.VMEM((EKP,), jnp.int32),
            pltpu.VMEM((EKP,), jnp.int32),
            pltpu.VMEM((EKP,), jnp.int32),
            pltpu.VMEM((EKP,), jnp.int32),
            pltpu.VMEM((EKP,), jnp.int32),
            pltpu.VMEM((EKP,), jnp.int32),
            pltpu.VMEM((EKP,), jnp.int32),
            pltpu.VMEM((EKP,), jnp.int32),
            pltpu.VMEM((EKP,), jnp.float32),
            pltpu.VMEM((EKP,), jnp.float32),
            pltpu.VMEM((EKP,), jnp.float32),
            pltpu.VMEM((EKP,), jnp.float32),
            pltpu.VMEM((FIN,), jnp.int32),
            pltpu.VMEM((C,), jnp.float32),
            pltpu.SemaphoreType.DMA,
            pltpu.SemaphoreType.DMA,
            pltpu.SemaphoreType.DMA,
            pltpu.SemaphoreType.DMA,
            pltpu.SemaphoreType.DMA,
            pltpu.SemaphoreType.DMA,
            pltpu.SemaphoreType.DMA,
            pltpu.SemaphoreType.DMA,
            pltpu.SemaphoreType.DMA,
            pltpu.SemaphoreType.DMA,
        ],
    )


_prop_relu = _make_prop("relu")
_prop_sigmoid = _make_prop("sigmoid")


def kernel(X, edge_index, edge_weight, W1, b1, W2, b2):
    row = edge_index[0].astype(jnp.int32)
    col = edge_index[1].astype(jnp.int32)
    loop = jnp.arange(N, dtype=jnp.int32)
    npad = E_PAD - E_ALL
    padi = jnp.arange(npad, dtype=jnp.int32) % N
    row_all = jnp.concatenate([row, loop, padi])
    col_all = jnp.concatenate([col, loop, padi])
    ew_all = jnp.concatenate([
        edge_weight,
        jnp.ones((N,), jnp.float32),
        jnp.zeros((npad,), jnp.float32),
    ])

    norm = _norm_kernel(row_all, col_all, ew_all)

    x2d = X.reshape(N * T, C)
    xw1 = _mm(x2d, W1)
    a1 = _prop_relu(xw1, row_all, col_all, norm, b1)
    xw2 = _mm(a1, W2)
    a2 = _prop_sigmoid(xw2, row_all, col_all, norm, b2)

    out = a2.reshape(N_PAD, T, C)[:N]
    return out[None]


# speedup vs baseline: 16.0793x; 1.3286x over previous
import functools

import jax
import jax.numpy as jnp
from jax import lax
from jax.experimental import pallas as pl
from jax.experimental.pallas import tpu as pltpu
from jax.experimental.pallas import tpu_sc as plsc

N = 10000
E = 160000
T = 12
C = 128

NC, NS, L = 2, 16, 16
NW = NC * NS
N_PAD = 10240
STRIPE = N_PAD // NS
E_ALL = E + N
E_PAD = 172032
EK = 128
CH_PER_W = E_PAD // (NW * EK)
EKP = 112
CHP = E_PAD // (NS * EKP)
NCH = E_PAD // EKP
FIN = 64
T_PER_CORE = T // NC
R_OUT = N_PAD * T

_MESH = plsc.VectorSubcoreMesh(core_axis_name="c", subcore_axis_name="s")
_SC_PARAMS = pltpu.CompilerParams(needs_layout_passes=False)


def _mm(x, w):
    R = x.shape[0]
    BM = 960
    assert R % BM == 0

    def body(x_ref, w_ref, o_ref):
        o_ref[...] = lax.dot_general(
            x_ref[...], w_ref[...], (((1,), (1,)), ((), ())),
            preferred_element_type=jnp.float32)

    return pl.pallas_call(
        body,
        grid=(R // BM,),
        in_specs=[
            pl.BlockSpec((BM, C), lambda i: (i, 0)),
            pl.BlockSpec((C, C), lambda i: (0, 0)),
        ],
        out_specs=pl.BlockSpec((BM, C), lambda i: (i, 0)),
        out_shape=jax.ShapeDtypeStruct((R, C), jnp.float32),
    )(x, w)


def _mm_bias_relu(x, w, b):
    R = x.shape[0]
    BM = 960
    assert R % BM == 0

    def body(x_ref, w_ref, b_ref, o_ref):
        xb = jnp.maximum(x_ref[...] + b_ref[...], 0.0)
        o_ref[...] = lax.dot_general(
            xb, w_ref[...], (((1,), (1,)), ((), ())),
            preferred_element_type=jnp.float32)

    return pl.pallas_call(
        body,
        grid=(R // BM,),
        in_specs=[
            pl.BlockSpec((BM, C), lambda i: (i, 0)),
            pl.BlockSpec((C, C), lambda i: (0, 0)),
            pl.BlockSpec((1, C), lambda i: (0, 0)),
        ],
        out_specs=pl.BlockSpec((BM, C), lambda i: (i, 0)),
        out_shape=jax.ShapeDtypeStruct((R, C), jnp.float32),
    )(x, w, b.reshape(1, C))


def _bias_sigmoid(x, b):
    BM = 960

    def body(x_ref, b_ref, o_ref):
        o_ref[...] = 1.0 / (1.0 + jnp.exp(-(x_ref[...] + b_ref[...])))

    return pl.pallas_call(
        body,
        grid=(N * T // BM,),
        in_specs=[
            pl.BlockSpec((BM, C), lambda i: (i, 0)),
            pl.BlockSpec((1, C), lambda i: (0, 0)),
        ],
        out_specs=pl.BlockSpec((BM, C), lambda i: (i, 0)),
        out_shape=jax.ShapeDtypeStruct((N * T, C), jnp.float32),
    )(x, b.reshape(1, C))


@functools.partial(
    pl.kernel,
    out_type=jax.ShapeDtypeStruct((E_PAD,), jnp.float32),
    mesh=_MESH,
    compiler_params=_SC_PARAMS,
    scratch_types=[
        pltpu.VMEM_SHARED((N_PAD,), jnp.float32),
        pltpu.VMEM_SHARED((N_PAD,), jnp.float32),
        pltpu.VMEM((STRIPE,), jnp.float32),
        pltpu.VMEM((N_PAD,), jnp.float32),
        pltpu.VMEM((EK,), jnp.int32),
        pltpu.VMEM((EK,), jnp.int32),
        pltpu.VMEM((EK,), jnp.float32),
        pltpu.VMEM((EK,), jnp.float32),
    ],
)
def _norm_kernel(row_hbm, col_hbm, ew_hbm, norm_hbm,
                 deg_sh, dis_sh, degb, disfull, rowb, colb, ewb, normb):
    c = lax.axis_index("c")
    s = lax.axis_index("s")
    wid = s * NC + c

    def zero_body(i, _):
        degb[pl.ds(i * L, L)] = jnp.zeros((L,), jnp.float32)
        return 0
    lax.fori_loop(0, STRIPE // L, zero_body, 0)
    pltpu.sync_copy(degb, deg_sh.at[pl.ds(s * STRIPE, STRIPE)])
    plsc.subcore_barrier()

    def deg_chunk(ci, _):
        base = ci * EK
        pltpu.sync_copy(col_hbm.at[pl.ds(base, EK)], colb)
        pltpu.sync_copy(ew_hbm.at[pl.ds(base, EK)], ewb)
        pltpu.sync_copy(ewb, deg_sh.at[colb], add=True)
        return 0
    lax.fori_loop(s * CH_PER_W, (s + 1) * CH_PER_W, deg_chunk, 0)
    lax.fori_loop((s + NS) * CH_PER_W, (s + NS + 1) * CH_PER_W, deg_chunk, 0)
    plsc.subcore_barrier()

    pltpu.sync_copy(deg_sh.at[pl.ds(s * STRIPE, STRIPE)], degb)

    def rsqrt_body(i, _):
        sl = pl.ds(i * L, L)
        x = degb[sl]
        xi = lax.bitcast_convert_type(x, jnp.int32)
        yi = jnp.int32(0x5F3759DF) - (xi >> 1)
        y = lax.bitcast_convert_type(yi, jnp.float32)
        hx = x * 0.5
        for _ in range(3):
            y = y * (1.5 - hx * y * y)
        degb[sl] = y
        return 0
    lax.fori_loop(0, STRIPE // L, rsqrt_body, 0)
    pltpu.sync_copy(degb, dis_sh.at[pl.ds(s * STRIPE, STRIPE)])
    plsc.subcore_barrier()

    pltpu.sync_copy(dis_sh, disfull)

    def norm_chunk(ci, _):
        base = ci * EK
        pltpu.sync_copy(row_hbm.at[pl.ds(base, EK)], rowb)
        pltpu.sync_copy(col_hbm.at[pl.ds(base, EK)], colb)
        pltpu.sync_copy(ew_hbm.at[pl.ds(base, EK)], ewb)
        for j in range(EK // L):
            sl = pl.ds(j * L, L)
            dr = plsc.load_gather(disfull, [rowb[sl]])
            dc = plsc.load_gather(disfull, [colb[sl]])
            normb[sl] = dr * ewb[sl] * dc
        pltpu.sync_copy(normb, norm_hbm.at[pl.ds(base, EK)])
        return 0
    lax.fori_loop(wid * CH_PER_W, (wid + 1) * CH_PER_W, norm_chunk, 0)


def _prop_body(xs_hbm, ebuf_hbm, out_hbm,
               acc, gbuf0, gbuf1, gbuf2,
               eb0, eb1, eb2, eb3, rw0, rw1, rw2, rw3, cl0, cl1, cl2, cl3,
               oidxA, oidxB, isem0, isem1, isem2, isem3,
               gsem0, gsem1, gsem2, ssem0, ssem1, ssem2):
    c = lax.axis_index("c")
    s = lax.axis_index("s")
    iot = lax.iota(jnp.int32, L)
    cbase = s * CHP
    P = [(eb0, rw0, cl0, isem0), (eb1, rw1, cl1, isem1),
         (eb2, rw2, cl2, isem2), (eb3, rw3, cl3, isem3)]
    G = [(gbuf0, gsem0, ssem0), (gbuf1, gsem1, ssem1), (gbuf2, gsem2, ssem2)]

    def istart(ci, p):
        pltpu.async_copy(ebuf_hbm.at[cbase + ci], p[0], p[3])

    def iwait(ci, p):
        pltpu.make_async_copy(ebuf_hbm.at[cbase + ci], p[0], p[3]).wait()

    def scale(p, g):
        ebv, gbuf = p[0], g[0]

        def sc4(q, _):
            for e in range(4):
                k = q * 4 + e
                svi = plsc.load_gather(
                    ebv, [jnp.zeros((L,), jnp.int32) + (2 * EKP + k)])
                sv = lax.bitcast_convert_type(svi, jnp.float32)
                for j in range(C // L):
                    sl = pl.ds(j * L, L)
                    gbuf[k, sl] = gbuf[k, sl] * sv
            return 0
        lax.fori_loop(0, EKP // 4, sc4, 0)

    def slice_body(ts, _):
        t = c * T_PER_CORE + ts

        def gstart(p, g):
            ebv, rowb, colb = p[0], p[1], p[2]
            for j in range(EKP // L):
                sl = pl.ds(j * L, L)
                rowb[sl] = ebv[pl.ds(j * L, L)] * T + t
                colb[sl] = ebv[pl.ds(EKP + j * L, L)]
            pltpu.async_copy(xs_hbm.at[rowb], g[0], g[1])

        def gwait(p, g):
            pltpu.make_async_copy(xs_hbm.at[p[1]], g[0], g[1]).wait()

        def sstart(p, g):
            pltpu.async_copy(g[0], acc.at[p[2]], g[2], add=True)

        def swait(p, g):
            pltpu.make_async_copy(g[0], acc.at[p[2]], g[2]).wait()

        def zb_body(i, _):
            for j in range(C // L):
                gbuf0[i, pl.ds(j * L, L)] = jnp.zeros((L,), jnp.float32)
            return 0
        lax.fori_loop(0, FIN, zb_body, 0)
        for k in range(STRIPE // FIN):
            pltpu.async_copy(
                gbuf0.at[pl.ds(0, FIN)],
                acc.at[pl.ds(s * STRIPE + k * FIN, FIN)], gsem0)
        for k in range(STRIPE // FIN):
            pltpu.make_async_copy(
                gbuf0.at[pl.ds(0, FIN)],
                acc.at[pl.ds(s * STRIPE + k * FIN, FIN)], gsem0).wait()
        plsc.subcore_barrier()

        istart(0, P[0])
        istart(1, P[1])
        istart(2, P[2])
        iwait(0, P[0])
        gstart(P[0], G[0])
        iwait(1, P[1])
        gstart(P[1], G[1])

        NU = 12
        NIT = CHP // NU

        def run(k, _):
            base = NU * k
            for q in range(NU):
                i = base + q
                p, g = P[q % 4], G[q % 3]
                pm1, gm1 = P[(q - 1) % 4], G[(q - 1) % 3]
                p2 = P[(q + 2) % 4]
                gwait(p, g)
                scale(p, g)
                sstart(p, g)
                if q == 0:
                    @pl.when(k > 0)
                    def _(pm1=pm1, gm1=gm1):
                        swait(pm1, gm1)
                    istart(i + 3, pm1)
                    iwait(i + 2, p2)
                    gstart(p2, gm1)
                else:
                    swait(pm1, gm1)
                    if q <= 8:
                        istart(i + 3, pm1)
                    else:
                        @pl.when(k < NIT - 1)
                        def _(i=i, pm1=pm1):
                            istart(i + 3, pm1)
                    if q <= 9:
                        iwait(i + 2, p2)
                        gstart(p2, gm1)
                    else:
                        @pl.when(k < NIT - 1)
                        def _(i=i, p2=p2, gm1=gm1):
                            iwait(i + 2, p2)
                            gstart(p2, gm1)
            return 0
        lax.fori_loop(0, NIT, run, 0)
        swait(P[3], G[2])
        plsc.subcore_barrier()

        NF = STRIPE // FIN
        FB = [(gbuf0, oidxA, gsem0, ssem0), (gbuf1, oidxB, gsem1, ssem1)]

        def fin_in(k, f):
            pltpu.async_copy(acc.at[pl.ds(s * STRIPE + k * FIN, FIN)],
                             f[0].at[pl.ds(0, FIN)], f[2])

        def fin_in_wait(k, f):
            pltpu.make_async_copy(acc.at[pl.ds(s * STRIPE + k * FIN, FIN)],
                                  f[0].at[pl.ds(0, FIN)], f[2]).wait()

        def fin_out(f):
            pltpu.async_copy(f[0].at[pl.ds(0, FIN)], out_hbm.at[f[1]], f[3])

        def fin_out_wait(f):
            pltpu.make_async_copy(f[0].at[pl.ds(0, FIN)],
                                  out_hbm.at[f[1]], f[3]).wait()

        fin_in(0, FB[0])
        for k in range(NF):
            f = FB[k % 2]
            fin_in_wait(k, f)
            nbase = s * STRIPE + k * FIN
            for j in range(FIN // L):
                sl = pl.ds(j * L, L)
                f[1][sl] = (iot + (nbase + j * L)) * T + t
            if k + 1 < NF:
                fo = FB[(k + 1) % 2]
                if k >= 1:
                    fin_out_wait(fo)
                fin_in(k + 1, fo)
            fin_out(f)
        fin_out_wait(FB[(NF - 2) % 2])
        fin_out_wait(FB[(NF - 1) % 2])
        return 0
    lax.fori_loop(0, T_PER_CORE, slice_body, 0)


_prop = pl.kernel(
    _prop_body,
    out_type=jax.ShapeDtypeStruct((R_OUT, C), jnp.float32),
    mesh=_MESH,
    compiler_params=_SC_PARAMS,
    scratch_types=[
        pltpu.VMEM_SHARED((N_PAD, C), jnp.float32),
        pltpu.VMEM((EKP, C), jnp.float32),
        pltpu.VMEM((EKP, C), jnp.float32),
        pltpu.VMEM((EKP, C), jnp.float32),
        pltpu.VMEM((3 * EKP,), jnp.int32),
        pltpu.VMEM((3 * EKP,), jnp.int32),
        pltpu.VMEM((3 * EKP,), jnp.int32),
        pltpu.VMEM((3 * EKP,), jnp.int32),
        pltpu.VMEM((EKP,), jnp.int32),
        pltpu.VMEM((EKP,), jnp.int32),
        pltpu.VMEM((EKP,), jnp.int32),
        pltpu.VMEM((EKP,), jnp.int32),
        pltpu.VMEM((EKP,), jnp.int32),
        pltpu.VMEM((EKP,), jnp.int32),
        pltpu.VMEM((EKP,), jnp.int32),
        pltpu.VMEM((EKP,), jnp.int32),
        pltpu.VMEM((FIN,), jnp.int32),
        pltpu.VMEM((FIN,), jnp.int32),
        pltpu.SemaphoreType.DMA,
        pltpu.SemaphoreType.DMA,
        pltpu.SemaphoreType.DMA,
        pltpu.SemaphoreType.DMA,
        pltpu.SemaphoreType.DMA,
        pltpu.SemaphoreType.DMA,
        pltpu.SemaphoreType.DMA,
        pltpu.SemaphoreType.DMA,
        pltpu.SemaphoreType.DMA,
        pltpu.SemaphoreType.DMA,
    ],
)


def kernel(X, edge_index, edge_weight, W1, b1, W2, b2):
    row = edge_index[0].astype(jnp.int32)
    col = edge_index[1].astype(jnp.int32)
    loop = jnp.arange(N, dtype=jnp.int32)
    npad = E_PAD - E_ALL
    padi = jnp.arange(npad, dtype=jnp.int32) % N
    row_all = jnp.concatenate([row, loop, padi])
    col_all = jnp.concatenate([col, loop, padi])
    ew_all = jnp.concatenate([
        edge_weight,
        jnp.ones((N,), jnp.float32),
        jnp.zeros((npad,), jnp.float32),
    ])

    norm = _norm_kernel(row_all, col_all, ew_all)

    ebuf = jnp.concatenate([
        row_all.reshape(NCH, EKP),
        col_all.reshape(NCH, EKP),
        lax.bitcast_convert_type(norm, jnp.int32).reshape(NCH, EKP),
    ], axis=1)

    x2d = X.reshape(N * T, C)
    xw1 = _mm(x2d, W1)
    p1 = _prop(xw1, ebuf)
    xw2 = _mm_bias_relu(p1, W2, b1)
    p2 = _prop(xw2, ebuf)
    out = _bias_sigmoid(p2, b2)

    return out.reshape(N, T, C)[None]


# speedup vs baseline: 16.1854x; 1.0066x over previous
import functools

import jax
import jax.numpy as jnp
from jax import lax
from jax.experimental import pallas as pl
from jax.experimental.pallas import tpu as pltpu
from jax.experimental.pallas import tpu_sc as plsc

N = 10000
E = 160000
T = 12
C = 128

NC, NS, L = 2, 16, 16
NW = NC * NS
N_PAD = 10240
STRIPE = N_PAD // NS
E_ALL = E + N
E_PAD = 172032
EK = 128
CH_PER_W = E_PAD // (NW * EK)
EKP = 112
CHP = E_PAD // (NS * EKP)
NCH = E_PAD // EKP
FIN = 64
T_PER_CORE = T // NC
R_OUT = N_PAD * T

_MESH = plsc.VectorSubcoreMesh(core_axis_name="c", subcore_axis_name="s")
_SC_PARAMS = pltpu.CompilerParams(needs_layout_passes=False)


def _mm(x, w):
    R = x.shape[0]
    BM = 960
    assert R % BM == 0

    def body(x_ref, w_ref, o_ref):
        o_ref[...] = lax.dot_general(
            x_ref[...], w_ref[...], (((1,), (1,)), ((), ())),
            preferred_element_type=jnp.float32)

    return pl.pallas_call(
        body,
        grid=(R // BM,),
        in_specs=[
            pl.BlockSpec((BM, C), lambda i: (i, 0)),
            pl.BlockSpec((C, C), lambda i: (0, 0)),
        ],
        out_specs=pl.BlockSpec((BM, C), lambda i: (i, 0)),
        out_shape=jax.ShapeDtypeStruct((R, C), jnp.float32),
    )(x, w)


def _mm_bias_relu(x, w, b):
    R = x.shape[0]
    BM = 960
    assert R % BM == 0

    def body(x_ref, w_ref, b_ref, o_ref):
        xb = jnp.maximum(x_ref[...] + b_ref[...], 0.0)
        o_ref[...] = lax.dot_general(
            xb, w_ref[...], (((1,), (1,)), ((), ())),
            preferred_element_type=jnp.float32)

    return pl.pallas_call(
        body,
        grid=(R // BM,),
        in_specs=[
            pl.BlockSpec((BM, C), lambda i: (i, 0)),
            pl.BlockSpec((C, C), lambda i: (0, 0)),
            pl.BlockSpec((1, C), lambda i: (0, 0)),
        ],
        out_specs=pl.BlockSpec((BM, C), lambda i: (i, 0)),
        out_shape=jax.ShapeDtypeStruct((R, C), jnp.float32),
    )(x, w, b.reshape(1, C))


def _bias_sigmoid(x, b):
    BM = 960

    def body(x_ref, b_ref, o_ref):
        o_ref[...] = 1.0 / (1.0 + jnp.exp(-(x_ref[...] + b_ref[...])))

    return pl.pallas_call(
        body,
        grid=(N * T // BM,),
        in_specs=[
            pl.BlockSpec((BM, C), lambda i: (i, 0)),
            pl.BlockSpec((1, C), lambda i: (0, 0)),
        ],
        out_specs=pl.BlockSpec((BM, C), lambda i: (i, 0)),
        out_shape=jax.ShapeDtypeStruct((N * T, C), jnp.float32),
    )(x, b.reshape(1, C))


NCW = NCH // NW


@functools.partial(
    pl.kernel,
    out_type=jax.ShapeDtypeStruct((NCH, 3 * EKP), jnp.int32),
    mesh=_MESH,
    compiler_params=_SC_PARAMS,
    scratch_types=[
        pltpu.VMEM_SHARED((N_PAD,), jnp.float32),
        pltpu.VMEM_SHARED((N_PAD,), jnp.float32),
        pltpu.VMEM((STRIPE,), jnp.float32),
        pltpu.VMEM((N_PAD,), jnp.float32),
        pltpu.VMEM((3 * EKP,), jnp.int32),
        pltpu.VMEM((3 * EKP,), jnp.int32),
        pltpu.VMEM((3 * EKP,), jnp.int32),
        pltpu.VMEM((3 * EKP,), jnp.int32),
        pltpu.VMEM((EKP,), jnp.float32),
        pltpu.VMEM((EKP,), jnp.float32),
        pltpu.VMEM((EKP,), jnp.float32),
        pltpu.VMEM((EKP,), jnp.float32),
        pltpu.VMEM((EKP,), jnp.int32),
        pltpu.VMEM((EKP,), jnp.int32),
        pltpu.VMEM((EKP,), jnp.int32),
        pltpu.VMEM((EKP,), jnp.int32),
        pltpu.SemaphoreType.DMA,
        pltpu.SemaphoreType.DMA,
        pltpu.SemaphoreType.DMA,
        pltpu.SemaphoreType.DMA,
        pltpu.SemaphoreType.DMA,
        pltpu.SemaphoreType.DMA,
        pltpu.SemaphoreType.DMA,
        pltpu.SemaphoreType.DMA,
    ],
)
def _norm_kernel(row_hbm, col_hbm, ew_hbm, ebuf_hbm,
                 deg_sh, dis_sh, degb, disfull,
                 eb0, eb1, eb2, eb3, ew0, ew1, ew2, ew3,
                 cb0, cb1, cb2, cb3,
                 isem0, isem1, isem2, isem3, osem0, osem1, osem2, osem3):
    c = lax.axis_index("c")
    s = lax.axis_index("s")
    wid = s * NC + c
    P = [(eb0, ew0, isem0, osem0, cb0), (eb1, ew1, isem1, osem1, cb1),
         (eb2, ew2, isem2, osem2, cb2), (eb3, ew3, isem3, osem3, cb3)]

    def istart(ci, p):
        ebv, ewb, isem = p[0], p[1], p[2]
        base = ci * EKP
        pltpu.async_copy(row_hbm.at[pl.ds(base, EKP)],
                         ebv.at[pl.ds(0, EKP)], isem)
        pltpu.async_copy(col_hbm.at[pl.ds(base, EKP)],
                         ebv.at[pl.ds(EKP, EKP)], isem)
        pltpu.async_copy(ew_hbm.at[pl.ds(base, EKP)], ewb, isem)

    def iwait(ci, p):
        ebv, ewb, isem = p[0], p[1], p[2]
        base = ci * EKP
        pltpu.make_async_copy(row_hbm.at[pl.ds(base, EKP)],
                              ebv.at[pl.ds(0, EKP)], isem).wait()
        pltpu.make_async_copy(col_hbm.at[pl.ds(base, EKP)],
                              ebv.at[pl.ds(EKP, EKP)], isem).wait()
        pltpu.make_async_copy(ew_hbm.at[pl.ds(base, EKP)], ewb, isem).wait()

    def zero_body(i, _):
        degb[pl.ds(i * L, L)] = jnp.zeros((L,), jnp.float32)
        return 0
    lax.fori_loop(0, STRIPE // L, zero_body, 0)
    pltpu.sync_copy(degb, deg_sh.at[pl.ds(s * STRIPE, STRIPE)])
    plsc.subcore_barrier()

    dbase = s * CHP

    def cstart(ci, p):
        base = ci * EKP
        pltpu.async_copy(col_hbm.at[pl.ds(base, EKP)], p[4], p[2])
        pltpu.async_copy(ew_hbm.at[pl.ds(base, EKP)], p[1], p[2])

    def cwait(ci, p):
        base = ci * EKP
        pltpu.make_async_copy(col_hbm.at[pl.ds(base, EKP)], p[4], p[2]).wait()
        pltpu.make_async_copy(ew_hbm.at[pl.ds(base, EKP)], p[1], p[2]).wait()

    def dstart(ci, p):
        pltpu.async_copy(p[1], deg_sh.at[p[4]], p[3], add=True)

    def dwait(ci, p):
        pltpu.make_async_copy(p[1], deg_sh.at[p[4]], p[3]).wait()

    cstart(dbase + 0, P[0])
    cstart(dbase + 1, P[1])
    cstart(dbase + 2, P[2])

    def deg_quad(k, _):
        base = dbase + 4 * k
        for q in range(4):
            i = base + q
            p, pm1 = P[q % 4], P[(q - 1) % 4]
            cwait(i, p)
            dstart(i, p)
            if q == 0:
                @pl.when(k > 0)
                def _(pm1=pm1, i=i):
                    dwait(i - 1, pm1)
                cstart(i + 3, pm1)
            else:
                dwait(i - 1, pm1)

                @pl.when(k < CHP // 4 - 1)
                def _(i=i, pm1=pm1):
                    cstart(i + 3, pm1)
        return 0
    lax.fori_loop(0, CHP // 4, deg_quad, 0)
    dwait(dbase + CHP - 1, P[3])
    plsc.subcore_barrier()

    pltpu.sync_copy(deg_sh.at[pl.ds(s * STRIPE, STRIPE)], degb)

    def rsqrt_body(i, _):
        sl = pl.ds(i * L, L)
        x = degb[sl]
        xi = lax.bitcast_convert_type(x, jnp.int32)
        yi = jnp.int32(0x5F3759DF) - (xi >> 1)
        y = lax.bitcast_convert_type(yi, jnp.float32)
        hx = x * 0.5
        for _ in range(3):
            y = y * (1.5 - hx * y * y)
        degb[sl] = y
        return 0
    lax.fori_loop(0, STRIPE // L, rsqrt_body, 0)
    pltpu.sync_copy(degb, dis_sh.at[pl.ds(s * STRIPE, STRIPE)])
    plsc.subcore_barrier()

    pltpu.sync_copy(dis_sh, disfull)

    nbase = wid * NCW

    def compute(p):
        ebv, ewb = p[0], p[1]
        for j in range(EKP // L):
            sl = pl.ds(j * L, L)
            dr = plsc.load_gather(disfull, [ebv[sl]])
            dc = plsc.load_gather(disfull, [ebv[pl.ds(EKP + j * L, L)]])
            nrm = dr * ewb[sl] * dc
            ebv[pl.ds(2 * EKP + j * L, L)] = lax.bitcast_convert_type(
                nrm, jnp.int32)

    def ostart(ci, p):
        pltpu.async_copy(p[0], ebuf_hbm.at[ci], p[3])

    def owait(ci, p):
        pltpu.make_async_copy(p[0], ebuf_hbm.at[ci], p[3]).wait()

    istart(nbase + 0, P[0])
    istart(nbase + 1, P[1])
    istart(nbase + 2, P[2])

    def norm_quad(k, _):
        base = nbase + 4 * k
        for q in range(4):
            i = base + q
            p, pm1 = P[q % 4], P[(q - 1) % 4]
            iwait(i, p)
            compute(p)
            ostart(i, p)
            if q == 0:
                @pl.when(k > 0)
                def _(pm1=pm1, i=i):
                    owait(i - 1, pm1)
                istart(i + 3, pm1)
            else:
                owait(i - 1, pm1)

                @pl.when(jnp.logical_or(q < 1, k < NCW // 4 - 1))
                def _(i=i, pm1=pm1):
                    istart(i + 3, pm1)
        return 0
    lax.fori_loop(0, NCW // 4, norm_quad, 0)
    owait(nbase + NCW - 1, P[3])


def _prop_body(xs_hbm, ebuf_hbm, out_hbm,
               acc, gbuf0, gbuf1, gbuf2,
               eb0, eb1, eb2, eb3, rw0, rw1, rw2, rw3, cl0, cl1, cl2, cl3,
               oidxA, oidxB, isem0, isem1, isem2, isem3,
               gsem0, gsem1, gsem2, ssem0, ssem1, ssem2):
    c = lax.axis_index("c")
    s = lax.axis_index("s")
    iot = lax.iota(jnp.int32, L)
    cbase = s * CHP
    P = [(eb0, rw0, cl0, isem0), (eb1, rw1, cl1, isem1),
         (eb2, rw2, cl2, isem2), (eb3, rw3, cl3, isem3)]
    G = [(gbuf0, gsem0, ssem0), (gbuf1, gsem1, ssem1), (gbuf2, gsem2, ssem2)]

    def istart(ci, p):
        pltpu.async_copy(ebuf_hbm.at[cbase + ci], p[0], p[3])

    def iwait(ci, p):
        pltpu.make_async_copy(ebuf_hbm.at[cbase + ci], p[0], p[3]).wait()

    def scale(p, g):
        ebv, gbuf = p[0], g[0]

        def sc8(q, _):
            for e in range(8):
                k = q * 8 + e
                svi = plsc.load_gather(
                    ebv, [jnp.zeros((L,), jnp.int32) + (2 * EKP + k)])
                sv = lax.bitcast_convert_type(svi, jnp.float32)
                for j in range(C // L):
                    sl = pl.ds(j * L, L)
                    gbuf[k, sl] = gbuf[k, sl] * sv
            return 0
        lax.fori_loop(0, EKP // 8, sc8, 0)

    def slice_body(ts, _):
        t = c * T_PER_CORE + ts

        def gstart(p, g):
            ebv, rowb, colb = p[0], p[1], p[2]
            for j in range(EKP // L):
                sl = pl.ds(j * L, L)
                rowb[sl] = ebv[pl.ds(j * L, L)] * T + t
                colb[sl] = ebv[pl.ds(EKP + j * L, L)]
            pltpu.async_copy(xs_hbm.at[rowb], g[0], g[1])

        def gwait(p, g):
            pltpu.make_async_copy(xs_hbm.at[p[1]], g[0], g[1]).wait()

        def sstart(p, g):
            pltpu.async_copy(g[0], acc.at[p[2]], g[2], add=True)

        def swait(p, g):
            pltpu.make_async_copy(g[0], acc.at[p[2]], g[2]).wait()

        def zb_body(i, _):
            for j in range(C // L):
                gbuf0[i, pl.ds(j * L, L)] = jnp.zeros((L,), jnp.float32)
            return 0
        lax.fori_loop(0, FIN, zb_body, 0)
        for k in range(STRIPE // FIN):
            pltpu.async_copy(
                gbuf0.at[pl.ds(0, FIN)],
                acc.at[pl.ds(s * STRIPE + k * FIN, FIN)], gsem0)
        for k in range(STRIPE // FIN):
            pltpu.make_async_copy(
                gbuf0.at[pl.ds(0, FIN)],
                acc.at[pl.ds(s * STRIPE + k * FIN, FIN)], gsem0).wait()
        plsc.subcore_barrier()

        istart(0, P[0])
        istart(1, P[1])
        istart(2, P[2])
        iwait(0, P[0])
        gstart(P[0], G[0])
        iwait(1, P[1])
        gstart(P[1], G[1])

        NU = 12
        NIT = CHP // NU

        def run(k, _):
            base = NU * k
            for q in range(NU):
                i = base + q
                p, g = P[q % 4], G[q % 3]
                pm1, gm1 = P[(q - 1) % 4], G[(q - 1) % 3]
                p2 = P[(q + 2) % 4]
                gwait(p, g)
                scale(p, g)
                sstart(p, g)
                if q == 0:
                    @pl.when(k > 0)
                    def _(pm1=pm1, gm1=gm1):
                        swait(pm1, gm1)
                    istart(i + 3, pm1)
                    iwait(i + 2, p2)
                    gstart(p2, gm1)
                else:
                    swait(pm1, gm1)
                    if q <= 8:
                        istart(i + 3, pm1)
                    else:
                        @pl.when(k < NIT - 1)
                        def _(i=i, pm1=pm1):
                            istart(i + 3, pm1)
                    if q <= 9:
                        iwait(i + 2, p2)
                        gstart(p2, gm1)
                    else:
                        @pl.when(k < NIT - 1)
                        def _(i=i, p2=p2, gm1=gm1):
                            iwait(i + 2, p2)
                            gstart(p2, gm1)
            return 0
        lax.fori_loop(0, NIT, run, 0)
        swait(P[3], G[2])
        plsc.subcore_barrier()

        NF = STRIPE // FIN
        FB = [(gbuf0, oidxA, gsem0, ssem0), (gbuf1, oidxB, gsem1, ssem1)]

        def fin_in(k, f):
            pltpu.async_copy(acc.at[pl.ds(s * STRIPE + k * FIN, FIN)],
                             f[0].at[pl.ds(0, FIN)], f[2])

        def fin_in_wait(k, f):
            pltpu.make_async_copy(acc.at[pl.ds(s * STRIPE + k * FIN, FIN)],
                                  f[0].at[pl.ds(0, FIN)], f[2]).wait()

        def fin_out(f):
            pltpu.async_copy(f[0].at[pl.ds(0, FIN)], out_hbm.at[f[1]], f[3])

        def fin_out_wait(f):
            pltpu.make_async_copy(f[0].at[pl.ds(0, FIN)],
                                  out_hbm.at[f[1]], f[3]).wait()

        fin_in(0, FB[0])
        for k in range(NF):
            f = FB[k % 2]
            fin_in_wait(k, f)
            nbase = s * STRIPE + k * FIN
            for j in range(FIN // L):
                sl = pl.ds(j * L, L)
                f[1][sl] = (iot + (nbase + j * L)) * T + t
            if k + 1 < NF:
                fo = FB[(k + 1) % 2]
                if k >= 1:
                    fin_out_wait(fo)
                fin_in(k + 1, fo)
            fin_out(f)
        fin_out_wait(FB[(NF - 2) % 2])
        fin_out_wait(FB[(NF - 1) % 2])
        return 0
    lax.fori_loop(0, T_PER_CORE, slice_body, 0)


_prop = pl.kernel(
    _prop_body,
    out_type=jax.ShapeDtypeStruct((R_OUT, C), jnp.float32),
    mesh=_MESH,
    compiler_params=_SC_PARAMS,
    scratch_types=[
        pltpu.VMEM_SHARED((N_PAD, C), jnp.float32),
        pltpu.VMEM((EKP, C), jnp.float32),
        pltpu.VMEM((EKP, C), jnp.float32),
        pltpu.VMEM((EKP, C), jnp.float32),
        pltpu.VMEM((3 * EKP,), jnp.int32),
        pltpu.VMEM((3 * EKP,), jnp.int32),
        pltpu.VMEM((3 * EKP,), jnp.int32),
        pltpu.VMEM((3 * EKP,), jnp.int32),
        pltpu.VMEM((EKP,), jnp.int32),
        pltpu.VMEM((EKP,), jnp.int32),
        pltpu.VMEM((EKP,), jnp.int32),
        pltpu.VMEM((EKP,), jnp.int32),
        pltpu.VMEM((EKP,), jnp.int32),
        pltpu.VMEM((EKP,), jnp.int32),
        pltpu.VMEM((EKP,), jnp.int32),
        pltpu.VMEM((EKP,), jnp.int32),
        pltpu.VMEM((FIN,), jnp.int32),
        pltpu.VMEM((FIN,), jnp.int32),
        pltpu.SemaphoreType.DMA,
        pltpu.SemaphoreType.DMA,
        pltpu.SemaphoreType.DMA,
        pltpu.SemaphoreType.DMA,
        pltpu.SemaphoreType.DMA,
        pltpu.SemaphoreType.DMA,
        pltpu.SemaphoreType.DMA,
        pltpu.SemaphoreType.DMA,
        pltpu.SemaphoreType.DMA,
        pltpu.SemaphoreType.DMA,
    ],
)


def kernel(X, edge_index, edge_weight, W1, b1, W2, b2):
    row = edge_index[0].astype(jnp.int32)
    col = edge_index[1].astype(jnp.int32)
    loop = jnp.arange(N, dtype=jnp.int32)
    npad = E_PAD - E_ALL
    padi = jnp.arange(npad, dtype=jnp.int32) % N
    row_all = jnp.concatenate([row, loop, padi])
    col_all = jnp.concatenate([col, loop, padi])
    ew_all = jnp.concatenate([
        edge_weight,
        jnp.ones((N,), jnp.float32),
        jnp.zeros((npad,), jnp.float32),
    ])

    ebuf = _norm_kernel(row_all, col_all, ew_all)

    x2d = X.reshape(N * T, C)
    xw1 = _mm(x2d, W1)
    p1 = _prop(xw1, ebuf)
    xw2 = _mm_bias_relu(p1, W2, b1)
    p2 = _prop(xw2, ebuf)
    out = _bias_sigmoid(p2, b2)

    return out.reshape(N, T, C)[None]


# speedup vs baseline: 16.2189x; 1.0021x over previous
import functools

import jax
import jax.numpy as jnp
from jax import lax
from jax.experimental import pallas as pl
from jax.experimental.pallas import tpu as pltpu
from jax.experimental.pallas import tpu_sc as plsc

N = 10000
E = 160000
T = 12
C = 128

NC, NS, L = 2, 16, 16
NW = NC * NS
N_PAD = 10240
STRIPE = N_PAD // NS
E_ALL = E + N
E_PAD = 172032
EK = 128
CH_PER_W = E_PAD // (NW * EK)
EKP = 112
CHP = E_PAD // (NS * EKP)
NCH = E_PAD // EKP
FIN = 64
T_PER_CORE = T // NC
R_OUT = N_PAD * T

_MESH = plsc.VectorSubcoreMesh(core_axis_name="c", subcore_axis_name="s")
_SC_PARAMS = pltpu.CompilerParams(needs_layout_passes=False)


def _mm(x, w):
    R = x.shape[0]
    BM = 960
    assert R % BM == 0

    def body(x_ref, w_ref, o_ref):
        o_ref[...] = lax.dot_general(
            x_ref[...], w_ref[...], (((1,), (1,)), ((), ())),
            preferred_element_type=jnp.float32)

    return pl.pallas_call(
        body,
        grid=(R // BM,),
        in_specs=[
            pl.BlockSpec((BM, C), lambda i: (i, 0)),
            pl.BlockSpec((C, C), lambda i: (0, 0)),
        ],
        out_specs=pl.BlockSpec((BM, C), lambda i: (i, 0)),
        out_shape=jax.ShapeDtypeStruct((R, C), jnp.float32),
    )(x, w)


def _mm_bias_relu(x, w, b):
    R = x.shape[0]
    BM = 960
    assert R % BM == 0

    def body(x_ref, w_ref, b_ref, o_ref):
        xb = jnp.maximum(x_ref[...] + b_ref[...], 0.0)
        o_ref[...] = lax.dot_general(
            xb, w_ref[...], (((1,), (1,)), ((), ())),
            preferred_element_type=jnp.float32)

    return pl.pallas_call(
        body,
        grid=(R // BM,),
        in_specs=[
            pl.BlockSpec((BM, C), lambda i: (i, 0)),
            pl.BlockSpec((C, C), lambda i: (0, 0)),
            pl.BlockSpec((1, C), lambda i: (0, 0)),
        ],
        out_specs=pl.BlockSpec((BM, C), lambda i: (i, 0)),
        out_shape=jax.ShapeDtypeStruct((R, C), jnp.float32),
    )(x, w, b.reshape(1, C))


def _bias_sigmoid(x, b):
    BM = 960

    def body(x_ref, b_ref, o_ref):
        o_ref[...] = 1.0 / (1.0 + jnp.exp(-(x_ref[...] + b_ref[...])))

    return pl.pallas_call(
        body,
        grid=(N * T // BM,),
        in_specs=[
            pl.BlockSpec((BM, C), lambda i: (i, 0)),
            pl.BlockSpec((1, C), lambda i: (0, 0)),
        ],
        out_specs=pl.BlockSpec((BM, C), lambda i: (i, 0)),
        out_shape=jax.ShapeDtypeStruct((N * T, C), jnp.float32),
    )(x, b.reshape(1, C))


NCW = NCH // NW


@functools.partial(
    pl.kernel,
    out_type=jax.ShapeDtypeStruct((NCH, 3 * EKP), jnp.int32),
    mesh=_MESH,
    compiler_params=_SC_PARAMS,
    scratch_types=[
        pltpu.VMEM_SHARED((N_PAD,), jnp.float32),
        pltpu.VMEM_SHARED((N_PAD,), jnp.float32),
        pltpu.VMEM((STRIPE,), jnp.float32),
        pltpu.VMEM((N_PAD,), jnp.float32),
        pltpu.VMEM((3 * EKP,), jnp.int32),
        pltpu.VMEM((3 * EKP,), jnp.int32),
        pltpu.VMEM((3 * EKP,), jnp.int32),
        pltpu.VMEM((3 * EKP,), jnp.int32),
        pltpu.VMEM((EKP,), jnp.float32),
        pltpu.VMEM((EKP,), jnp.float32),
        pltpu.VMEM((EKP,), jnp.float32),
        pltpu.VMEM((EKP,), jnp.float32),
        pltpu.VMEM((EKP,), jnp.int32),
        pltpu.VMEM((EKP,), jnp.int32),
        pltpu.VMEM((EKP,), jnp.int32),
        pltpu.VMEM((EKP,), jnp.int32),
        pltpu.SemaphoreType.DMA,
        pltpu.SemaphoreType.DMA,
        pltpu.SemaphoreType.DMA,
        pltpu.SemaphoreType.DMA,
        pltpu.SemaphoreType.DMA,
        pltpu.SemaphoreType.DMA,
        pltpu.SemaphoreType.DMA,
        pltpu.SemaphoreType.DMA,
    ],
)
def _norm_kernel(row_hbm, col_hbm, ew_hbm, ebuf_hbm,
                 deg_sh, dis_sh, degb, disfull,
                 eb0, eb1, eb2, eb3, ew0, ew1, ew2, ew3,
                 cb0, cb1, cb2, cb3,
                 isem0, isem1, isem2, isem3, osem0, osem1, osem2, osem3):
    c = lax.axis_index("c")
    s = lax.axis_index("s")
    wid = s * NC + c
    P = [(eb0, ew0, isem0, osem0, cb0), (eb1, ew1, isem1, osem1, cb1),
         (eb2, ew2, isem2, osem2, cb2), (eb3, ew3, isem3, osem3, cb3)]

    def istart(ci, p):
        ebv, ewb, isem = p[0], p[1], p[2]
        base = ci * EKP
        pltpu.async_copy(row_hbm.at[pl.ds(base, EKP)],
                         ebv.at[pl.ds(0, EKP)], isem)
        pltpu.async_copy(col_hbm.at[pl.ds(base, EKP)],
                         ebv.at[pl.ds(EKP, EKP)], isem)
        pltpu.async_copy(ew_hbm.at[pl.ds(base, EKP)], ewb, isem)

    def iwait(ci, p):
        ebv, ewb, isem = p[0], p[1], p[2]
        base = ci * EKP
        pltpu.make_async_copy(row_hbm.at[pl.ds(base, EKP)],
                              ebv.at[pl.ds(0, EKP)], isem).wait()
        pltpu.make_async_copy(col_hbm.at[pl.ds(base, EKP)],
                              ebv.at[pl.ds(EKP, EKP)], isem).wait()
        pltpu.make_async_copy(ew_hbm.at[pl.ds(base, EKP)], ewb, isem).wait()

    def zero_body(i, _):
        degb[pl.ds(i * L, L)] = jnp.zeros((L,), jnp.float32)
        return 0
    lax.fori_loop(0, STRIPE // L, zero_body, 0)
    pltpu.sync_copy(degb, deg_sh.at[pl.ds(s * STRIPE, STRIPE)])
    plsc.subcore_barrier()

    dbase = s * CHP

    def cstart(ci, p):
        base = ci * EKP
        pltpu.async_copy(col_hbm.at[pl.ds(base, EKP)], p[4], p[2])
        pltpu.async_copy(ew_hbm.at[pl.ds(base, EKP)], p[1], p[2])

    def cwait(ci, p):
        base = ci * EKP
        pltpu.make_async_copy(col_hbm.at[pl.ds(base, EKP)], p[4], p[2]).wait()
        pltpu.make_async_copy(ew_hbm.at[pl.ds(base, EKP)], p[1], p[2]).wait()

    def dstart(ci, p):
        pltpu.async_copy(p[1], deg_sh.at[p[4]], p[3], add=True)

    def dwait(ci, p):
        pltpu.make_async_copy(p[1], deg_sh.at[p[4]], p[3]).wait()

    cstart(dbase + 0, P[0])
    cstart(dbase + 1, P[1])
    cstart(dbase + 2, P[2])

    def deg_quad(k, _):
        base = dbase + 4 * k
        for q in range(4):
            i = base + q
            p, pm1 = P[q % 4], P[(q - 1) % 4]
            cwait(i, p)
            dstart(i, p)
            if q == 0:
                @pl.when(k > 0)
                def _(pm1=pm1, i=i):
                    dwait(i - 1, pm1)
                cstart(i + 3, pm1)
            else:
                dwait(i - 1, pm1)

                @pl.when(k < CHP // 4 - 1)
                def _(i=i, pm1=pm1):
                    cstart(i + 3, pm1)
        return 0
    lax.fori_loop(0, CHP // 4, deg_quad, 0)
    dwait(dbase + CHP - 1, P[3])
    plsc.subcore_barrier()

    pltpu.sync_copy(deg_sh.at[pl.ds(s * STRIPE, STRIPE)], degb)

    def rsqrt_body(i, _):
        sl = pl.ds(i * L, L)
        x = degb[sl]
        xi = lax.bitcast_convert_type(x, jnp.int32)
        yi = jnp.int32(0x5F3759DF) - (xi >> 1)
        y = lax.bitcast_convert_type(yi, jnp.float32)
        hx = x * 0.5
        for _ in range(3):
            y = y * (1.5 - hx * y * y)
        degb[sl] = y
        return 0
    lax.fori_loop(0, STRIPE // L, rsqrt_body, 0)
    pltpu.sync_copy(degb, dis_sh.at[pl.ds(s * STRIPE, STRIPE)])
    plsc.subcore_barrier()

    pltpu.sync_copy(dis_sh, disfull)

    nbase = wid * NCW

    def compute(p):
        ebv, ewb = p[0], p[1]
        for j in range(EKP // L):
            sl = pl.ds(j * L, L)
            dr = plsc.load_gather(disfull, [ebv[sl]])
            dc = plsc.load_gather(disfull, [ebv[pl.ds(EKP + j * L, L)]])
            nrm = dr * ewb[sl] * dc
            ebv[pl.ds(2 * EKP + j * L, L)] = lax.bitcast_convert_type(
                nrm, jnp.int32)

    def ostart(ci, p):
        pltpu.async_copy(p[0], ebuf_hbm.at[ci], p[3])

    def owait(ci, p):
        pltpu.make_async_copy(p[0], ebuf_hbm.at[ci], p[3]).wait()

    istart(nbase + 0, P[0])
    istart(nbase + 1, P[1])
    istart(nbase + 2, P[2])

    def norm_quad(k, _):
        base = nbase + 4 * k
        for q in range(4):
            i = base + q
            p, pm1 = P[q % 4], P[(q - 1) % 4]
            iwait(i, p)
            compute(p)
            ostart(i, p)
            if q == 0:
                @pl.when(k > 0)
                def _(pm1=pm1, i=i):
                    owait(i - 1, pm1)
                istart(i + 3, pm1)
            else:
                owait(i - 1, pm1)

                @pl.when(jnp.logical_or(q < 1, k < NCW // 4 - 1))
                def _(i=i, pm1=pm1):
                    istart(i + 3, pm1)
        return 0
    lax.fori_loop(0, NCW // 4, norm_quad, 0)
    owait(nbase + NCW - 1, P[3])


SPL = 64


def _prop_body(xs_hbm, ebuf_hbm, out_hbm,
               acc, gbuf0, gbuf1, gbuf2,
               eb0, eb1, eb2, eb3, rw0, rw1, rw2, rw3,
               ca0, ca1, ca2, ca3, cb0, cb1, cb2, cb3,
               oidxA, oidxB, isem0, isem1, isem2, isem3,
               gsem0, gsem1, gsem2,
               ssem0, ssem1, ssem2, tsem0, tsem1, tsem2):
    c = lax.axis_index("c")
    s = lax.axis_index("s")
    iot = lax.iota(jnp.int32, L)
    cbase = s * CHP
    P = [(eb0, rw0, ca0, isem0, cb0), (eb1, rw1, ca1, isem1, cb1),
         (eb2, rw2, ca2, isem2, cb2), (eb3, rw3, ca3, isem3, cb3)]
    G = [(gbuf0, gsem0, ssem0, tsem0), (gbuf1, gsem1, ssem1, tsem1),
         (gbuf2, gsem2, ssem2, tsem2)]

    def istart(ci, p):
        pltpu.async_copy(ebuf_hbm.at[cbase + ci], p[0], p[3])

    def iwait(ci, p):
        pltpu.make_async_copy(ebuf_hbm.at[cbase + ci], p[0], p[3]).wait()

    def scale(p, g):
        ebv, gbuf = p[0], g[0]

        def sc8(q, _):
            for e in range(8):
                k = q * 8 + e
                svi = plsc.load_gather(
                    ebv, [jnp.zeros((L,), jnp.int32) + (2 * EKP + k)])
                sv = lax.bitcast_convert_type(svi, jnp.float32)
                for j in range(C // L):
                    sl = pl.ds(j * L, L)
                    gbuf[k, sl] = gbuf[k, sl] * sv
            return 0
        lax.fori_loop(0, EKP // 8, sc8, 0)

    def slice_body(ts, _):
        t = c * T_PER_CORE + ts

        def gstart(p, g):
            ebv, rowb, cola, colb = p[0], p[1], p[2], p[4]
            for j in range(EKP // L):
                sl = pl.ds(j * L, L)
                rowb[sl] = ebv[pl.ds(j * L, L)] * T + t
            for j in range(SPL // L):
                cola[pl.ds(j * L, L)] = ebv[pl.ds(EKP + j * L, L)]
            for j in range((EKP - SPL) // L):
                colb[pl.ds(j * L, L)] = ebv[pl.ds(EKP + SPL + j * L, L)]
            pltpu.async_copy(xs_hbm.at[rowb], g[0], g[1])

        def gwait(p, g):
            pltpu.make_async_copy(xs_hbm.at[p[1]], g[0], g[1]).wait()

        def sstart(p, g):
            pltpu.async_copy(g[0].at[pl.ds(0, SPL)], acc.at[p[2]], g[2],
                             add=True)
            pltpu.async_copy(g[0].at[pl.ds(SPL, EKP - SPL)], acc.at[p[4]],
                             g[3], add=True)

        def swait(p, g):
            pltpu.make_async_copy(g[0].at[pl.ds(0, SPL)], acc.at[p[2]],
                                  g[2]).wait()
            pltpu.make_async_copy(g[0].at[pl.ds(SPL, EKP - SPL)],
                                  acc.at[p[4]], g[3]).wait()

        def zb_body(i, _):
            for j in range(C // L):
                gbuf0[i, pl.ds(j * L, L)] = jnp.zeros((L,), jnp.float32)
            return 0
        lax.fori_loop(0, FIN, zb_body, 0)
        for k in range(STRIPE // FIN):
            pltpu.async_copy(
                gbuf0.at[pl.ds(0, FIN)],
                acc.at[pl.ds(s * STRIPE + k * FIN, FIN)], gsem0)
        for k in range(STRIPE // FIN):
            pltpu.make_async_copy(
                gbuf0.at[pl.ds(0, FIN)],
                acc.at[pl.ds(s * STRIPE + k * FIN, FIN)], gsem0).wait()
        plsc.subcore_barrier()

        istart(0, P[0])
        istart(1, P[1])
        istart(2, P[2])
        iwait(0, P[0])
        gstart(P[0], G[0])
        iwait(1, P[1])
        gstart(P[1], G[1])

        NU = 12
        NIT = CHP // NU

        def run(k, _):
            base = NU * k
            for q in range(NU):
                i = base + q
                p, g = P[q % 4], G[q % 3]
                pm1, gm1 = P[(q - 1) % 4], G[(q - 1) % 3]
                p2 = P[(q + 2) % 4]
                gwait(p, g)
                scale(p, g)
                sstart(p, g)
                if q == 0:
                    @pl.when(k > 0)
                    def _(pm1=pm1, gm1=gm1):
                        swait(pm1, gm1)
                    istart(i + 3, pm1)
                    iwait(i + 2, p2)
                    gstart(p2, gm1)
                else:
                    swait(pm1, gm1)
                    if q <= 8:
                        istart(i + 3, pm1)
                    else:
                        @pl.when(k < NIT - 1)
                        def _(i=i, pm1=pm1):
                            istart(i + 3, pm1)
                    if q <= 9:
                        iwait(i + 2, p2)
                        gstart(p2, gm1)
                    else:
                        @pl.when(k < NIT - 1)
                        def _(i=i, p2=p2, gm1=gm1):
                            iwait(i + 2, p2)
                            gstart(p2, gm1)
            return 0
        lax.fori_loop(0, NIT, run, 0)
        swait(P[3], G[2])
        plsc.subcore_barrier()

        NF = STRIPE // FIN
        FB = [(gbuf0, oidxA, gsem0, ssem0), (gbuf1, oidxB, gsem1, ssem1)]

        def fin_in(k, f):
            pltpu.async_copy(acc.at[pl.ds(s * STRIPE + k * FIN, FIN)],
                             f[0].at[pl.ds(0, FIN)], f[2])

        def fin_in_wait(k, f):
            pltpu.make_async_copy(acc.at[pl.ds(s * STRIPE + k * FIN, FIN)],
                                  f[0].at[pl.ds(0, FIN)], f[2]).wait()

        def fin_out(f):
            pltpu.async_copy(f[0].at[pl.ds(0, FIN)], out_hbm.at[f[1]], f[3])

        def fin_out_wait(f):
            pltpu.make_async_copy(f[0].at[pl.ds(0, FIN)],
                                  out_hbm.at[f[1]], f[3]).wait()

        fin_in(0, FB[0])
        for k in range(NF):
            f = FB[k % 2]
            fin_in_wait(k, f)
            nbase = s * STRIPE + k * FIN
            for j in range(FIN // L):
                sl = pl.ds(j * L, L)
                f[1][sl] = (iot + (nbase + j * L)) * T + t
            if k + 1 < NF:
                fo = FB[(k + 1) % 2]
                if k >= 1:
                    fin_out_wait(fo)
                fin_in(k + 1, fo)
            fin_out(f)
        fin_out_wait(FB[(NF - 2) % 2])
        fin_out_wait(FB[(NF - 1) % 2])
        return 0
    lax.fori_loop(0, T_PER_CORE, slice_body, 0)


_prop = pl.kernel(
    _prop_body,
    out_type=jax.ShapeDtypeStruct((R_OUT, C), jnp.float32),
    mesh=_MESH,
    compiler_params=_SC_PARAMS,
    scratch_types=[
        pltpu.VMEM_SHARED((N_PAD, C), jnp.float32),
        pltpu.VMEM((EKP, C), jnp.float32),
        pltpu.VMEM((EKP, C), jnp.float32),
        pltpu.VMEM((EKP, C), jnp.float32),
        pltpu.VMEM((3 * EKP,), jnp.int32),
        pltpu.VMEM((3 * EKP,), jnp.int32),
        pltpu.VMEM((3 * EKP,), jnp.int32),
        pltpu.VMEM((3 * EKP,), jnp.int32),
        pltpu.VMEM((EKP,), jnp.int32),
        pltpu.VMEM((EKP,), jnp.int32),
        pltpu.VMEM((EKP,), jnp.int32),
        pltpu.VMEM((EKP,), jnp.int32),
        pltpu.VMEM((SPL,), jnp.int32),
        pltpu.VMEM((SPL,), jnp.int32),
        pltpu.VMEM((SPL,), jnp.int32),
        pltpu.VMEM((SPL,), jnp.int32),
        pltpu.VMEM((EKP - SPL,), jnp.int32),
        pltpu.VMEM((EKP - SPL,), jnp.int32),
        pltpu.VMEM((EKP - SPL,), jnp.int32),
        pltpu.VMEM((EKP - SPL,), jnp.int32),
        pltpu.VMEM((FIN,), jnp.int32),
        pltpu.VMEM((FIN,), jnp.int32),
        pltpu.SemaphoreType.DMA,
        pltpu.SemaphoreType.DMA,
        pltpu.SemaphoreType.DMA,
        pltpu.SemaphoreType.DMA,
        pltpu.SemaphoreType.DMA,
        pltpu.SemaphoreType.DMA,
        pltpu.SemaphoreType.DMA,
        pltpu.SemaphoreType.DMA,
        pltpu.SemaphoreType.DMA,
        pltpu.SemaphoreType.DMA,
        pltpu.SemaphoreType.DMA,
        pltpu.SemaphoreType.DMA,
        pltpu.SemaphoreType.DMA,
    ],
)


def kernel(X, edge_index, edge_weight, W1, b1, W2, b2):
    row = edge_index[0].astype(jnp.int32)
    col = edge_index[1].astype(jnp.int32)
    loop = jnp.arange(N, dtype=jnp.int32)
    npad = E_PAD - E_ALL
    padi = jnp.arange(npad, dtype=jnp.int32) % N
    row_all = jnp.concatenate([row, loop, padi])
    col_all = jnp.concatenate([col, loop, padi])
    ew_all = jnp.concatenate([
        edge_weight,
        jnp.ones((N,), jnp.float32),
        jnp.zeros((npad,), jnp.float32),
    ])

    ebuf = _norm_kernel(row_all, col_all, ew_all)

    x2d = X.reshape(N * T, C)
    xw1 = _mm(x2d, W1)
    p1 = _prop(xw1, ebuf)
    xw2 = _mm_bias_relu(p1, W2, b1)
    p2 = _prop(xw2, ebuf)
    out = _bias_sigmoid(p2, b2)

    return out.reshape(N, T, C)[None]
